# baseline (speedup 1.0000x reference)
import functools

import jax
import jax.numpy as jnp
from jax import lax
from jax.experimental import pallas as pl
from jax.experimental.pallas import tpu as pltpu
from jax.experimental.pallas import tpu_sc as plsc

N = 10000
E = 320000
D = 64

_INTERPRET = False


def _mlp2_body(x_ref, w1_ref, b1_ref, w2_ref, b2_ref, o_ref):
    h = jnp.maximum(
        jnp.dot(x_ref[...], w1_ref[...], preferred_element_type=jnp.float32)
        + b1_ref[...], 0.0)
    o_ref[...] = (
        jnp.dot(h, w2_ref[...], preferred_element_type=jnp.float32)
        + b2_ref[...])


def _mlp2(x, w1, b1, w2, b2, blk):
    rows = x.shape[0]
    f_in, f_mid = w1.shape
    f_out = w2.shape[1]
    grid = rows // blk
    return pl.pallas_call(
        _mlp2_body,
        grid=(grid,),
        in_specs=[
            pl.BlockSpec((blk, f_in), lambda i: (i, 0)),
            pl.BlockSpec((f_in, f_mid), lambda i: (0, 0)),
            pl.BlockSpec((1, f_mid), lambda i: (0, 0)),
            pl.BlockSpec((f_mid, f_out), lambda i: (0, 0)),
            pl.BlockSpec((1, f_out), lambda i: (0, 0)),
        ],
        out_specs=pl.BlockSpec((blk, f_out), lambda i: (i, 0)),
        out_shape=jax.ShapeDtypeStruct((rows, f_out), jnp.float32),
        interpret=_INTERPRET,
    )(x, w1, b1.reshape(1, -1), w2, b2.reshape(1, -1))


def _matmul_bias_body(x_ref, w_ref, b_ref, o_ref):
    o_ref[...] = (
        jnp.dot(x_ref[...], w_ref[...], preferred_element_type=jnp.float32)
        + b_ref[...])


def _matmul_bias(x, w, b, blk):
    rows = x.shape[0]
    f_in, f_out = w.shape
    grid = rows // blk
    return pl.pallas_call(
        _matmul_bias_body,
        grid=(grid,),
        in_specs=[
            pl.BlockSpec((blk, f_in), lambda i: (i, 0)),
            pl.BlockSpec((f_in, f_out), lambda i: (0, 0)),
            pl.BlockSpec((1, f_out), lambda i: (0, 0)),
        ],
        out_specs=pl.BlockSpec((blk, f_out), lambda i: (i, 0)),
        out_shape=jax.ShapeDtypeStruct((rows, f_out), jnp.float32),
        interpret=_INTERPRET,
    )(x, w, b.reshape(1, -1))


def _node_update_body(h_ref, ah_ref, nd_ref, o_ref):
    num = nd_ref[0, :, :D] + nd_ref[1, :, :D]
    den = nd_ref[0, :, D:] + nd_ref[1, :, D:] + 1e-6
    o_ref[...] = h_ref[...] + jnp.maximum(ah_ref[...] + num / den, 0.0)


def _node_update(h, ah, ndpart, blk):
    grid = N // blk
    return pl.pallas_call(
        _node_update_body,
        grid=(grid,),
        in_specs=[
            pl.BlockSpec((blk, D), lambda i: (i, 0)),
            pl.BlockSpec((blk, D), lambda i: (i, 0)),
            pl.BlockSpec((2, blk, 2 * D), lambda i: (0, i, 0)),
        ],
        out_specs=pl.BlockSpec((blk, D), lambda i: (i, 0)),
        out_shape=jax.ShapeDtypeStruct((N, D), jnp.float32),
        interpret=_INTERPRET,
    )(h, ah, ndpart)


def _edge_pass(ce, ee, bd, eh_tab, src, dst):
    bde = bd[src]
    e_hat = ce + bde[:, D:] + eh_tab[dst]
    sigma = jax.nn.sigmoid(e_hat)
    ee_new = ee + jnp.maximum(e_hat, 0.0)
    contrib = jnp.concatenate([sigma * bde[:, :D], sigma], axis=1)
    nd = jax.ops.segment_sum(contrib, dst, num_segments=N)
    ndpart = jnp.stack([nd, jnp.zeros_like(nd)])
    return ee_new, ndpart


def _final_gather(p, ha_hb, src, dst):
    return jnp.maximum(p + ha_hb[src, :D] + ha_hb[dst, D:], 0.0)


def kernel(edge_index, x, e, params):
    src = edge_index[0]
    dst = edge_index[1]
    p = params

    h = _mlp2(x, p['enc_W1'], p['enc_b1'], p['enc_W2'], p['enc_b2'], 1000)
    ee = _mlp2(e, p['e1_W'], p['e1_b'], p['e2_W'], p['e2_b'], 2000)

    for layer in p['gnn']:
        wcat = jnp.concatenate(
            [layer['A_W'], layer['B_W'], layer['D_W'], layer['E_W']], axis=1)
        bcat = jnp.concatenate(
            [layer['A_b'], layer['B_b'], layer['D_b'], layer['E_b']])
        abde = _matmul_bias(h, wcat, bcat, 1000)
        ce = _matmul_bias(ee, layer['C_W'], layer['C_b'], 2000)
        bd = abde[:, D:3 * D]
        eh_tab = abde[:, 3 * D:]
        ee, ndpart = _edge_pass(ce, ee, bd, eh_tab, src, dst)
        h = _node_update(h, abde[:, :D], ndpart, 1000)

    w1 = p['sp_W1']
    hahb = _matmul_bias(
        h, jnp.concatenate([w1[:D], w1[D:2 * D]], axis=1),
        jnp.zeros((2 * D,), jnp.float32), 1000)
    pe = _matmul_bias(ee, w1[2 * D:], p['sp_b1'], 2000)
    q = _final_gather(pe, hahb, src, dst)
    scores = _matmul_bias(q, p['sp_W2'], p['sp_b2'], 2000)
    return scores


# speedup vs baseline: 139.5632x; 139.5632x over previous
import functools

import jax
import jax.numpy as jnp
from jax import lax
from jax.experimental import pallas as pl
from jax.experimental.pallas import tpu as pltpu
from jax.experimental.pallas import tpu_sc as plsc

N = 10000
E = 320000
D = 64

_INTERPRET = False


def _mlp2_body(x_ref, w1_ref, b1_ref, w2_ref, b2_ref, o_ref):
    h = jnp.maximum(
        jnp.dot(x_ref[...], w1_ref[...], preferred_element_type=jnp.float32)
        + b1_ref[...], 0.0)
    o_ref[...] = (
        jnp.dot(h, w2_ref[...], preferred_element_type=jnp.float32)
        + b2_ref[...])


def _mlp2(x, w1, b1, w2, b2, blk):
    rows = x.shape[0]
    f_in, f_mid = w1.shape
    f_out = w2.shape[1]
    grid = rows // blk
    return pl.pallas_call(
        _mlp2_body,
        grid=(grid,),
        in_specs=[
            pl.BlockSpec((blk, f_in), lambda i: (i, 0)),
            pl.BlockSpec((f_in, f_mid), lambda i: (0, 0)),
            pl.BlockSpec((1, f_mid), lambda i: (0, 0)),
            pl.BlockSpec((f_mid, f_out), lambda i: (0, 0)),
            pl.BlockSpec((1, f_out), lambda i: (0, 0)),
        ],
        out_specs=pl.BlockSpec((blk, f_out), lambda i: (i, 0)),
        out_shape=jax.ShapeDtypeStruct((rows, f_out), jnp.float32),
        interpret=_INTERPRET,
    )(x, w1, b1.reshape(1, -1), w2, b2.reshape(1, -1))


def _matmul_bias_body(x_ref, w_ref, b_ref, o_ref):
    o_ref[...] = (
        jnp.dot(x_ref[...], w_ref[...], preferred_element_type=jnp.float32)
        + b_ref[...])


def _matmul_bias(x, w, b, blk):
    rows = x.shape[0]
    f_in, f_out = w.shape
    grid = rows // blk
    return pl.pallas_call(
        _matmul_bias_body,
        grid=(grid,),
        in_specs=[
            pl.BlockSpec((blk, f_in), lambda i: (i, 0)),
            pl.BlockSpec((f_in, f_out), lambda i: (0, 0)),
            pl.BlockSpec((1, f_out), lambda i: (0, 0)),
        ],
        out_specs=pl.BlockSpec((blk, f_out), lambda i: (i, 0)),
        out_shape=jax.ShapeDtypeStruct((rows, f_out), jnp.float32),
        interpret=_INTERPRET,
    )(x, w, b.reshape(1, -1))


def _node_mm2_body(x_ref, w_ref, b_ref, o1_ref, o2_ref):
    o = (jnp.dot(x_ref[...], w_ref[...], preferred_element_type=jnp.float32)
         + b_ref[...])
    o1_ref[...] = o[:, :2 * D]
    o2_ref[...] = o[:, 2 * D:]


def _node_mm2(x, w, b, blk):
    grid = N // blk
    return pl.pallas_call(
        _node_mm2_body,
        grid=(grid,),
        in_specs=[
            pl.BlockSpec((blk, D), lambda i: (i, 0)),
            pl.BlockSpec((D, 4 * D), lambda i: (0, 0)),
            pl.BlockSpec((1, 4 * D), lambda i: (0, 0)),
        ],
        out_specs=[
            pl.BlockSpec((blk, 2 * D), lambda i: (i, 0)),
            pl.BlockSpec((blk, 2 * D), lambda i: (i, 0)),
        ],
        out_shape=[
            jax.ShapeDtypeStruct((N, 2 * D), jnp.float32),
            jax.ShapeDtypeStruct((N, 2 * D), jnp.float32),
        ],
        interpret=_INTERPRET,
    )(x, w, b.reshape(1, -1))


def _node_update_body(h_ref, ea_ref, nd_ref, o_ref):
    num = nd_ref[0, :, :D] + nd_ref[1, :, :D]
    den = nd_ref[0, :, D:] + nd_ref[1, :, D:] + 1e-6
    o_ref[...] = h_ref[...] + jnp.maximum(
        ea_ref[:, D:] + num / den, 0.0)


def _node_update(h, ea, ndpart, blk):
    grid = N // blk
    return pl.pallas_call(
        _node_update_body,
        grid=(grid,),
        in_specs=[
            pl.BlockSpec((blk, D), lambda i: (i, 0)),
            pl.BlockSpec((blk, 2 * D), lambda i: (i, 0)),
            pl.BlockSpec((2, blk, 2 * D), lambda i: (0, i, 0)),
        ],
        out_specs=pl.BlockSpec((blk, D), lambda i: (i, 0)),
        out_shape=jax.ShapeDtypeStruct((N, D), jnp.float32),
        interpret=_INTERPRET,
    )(h, ea, ndpart)


_NC = 2
_NS = 16
_NW = _NC * _NS
_B = 40
_EPW = E // _NW
_TPW = _EPW // _B
_NCH = N // _B


def _sc_mesh():
    return plsc.VectorSubcoreMesh(
        core_axis_name="c", subcore_axis_name="s",
        num_cores=_NC, num_subcores=_NS)


def _sc_layer(src, dst, ce, ee, bd, ea):
    @functools.partial(
        pl.kernel,
        out_type=[jax.ShapeDtypeStruct((E, D), jnp.float32),
                  jax.ShapeDtypeStruct((_NC, N, 2 * D), jnp.float32)],
        mesh=_sc_mesh(),
        scratch_types=[
            pltpu.VMEM((_B,), jnp.int32),
            pltpu.VMEM((_B,), jnp.int32),
            pltpu.VMEM((_B, 2 * D), jnp.float32),
            pltpu.VMEM((_B, 2 * D), jnp.float32),
            pltpu.VMEM((_B, D), jnp.float32),
            pltpu.VMEM((_B, D), jnp.float32),
            pltpu.VMEM((_B, D), jnp.float32),
            pltpu.VMEM((_B, 2 * D), jnp.float32),
            pltpu.VMEM_SHARED((N, 2 * D), jnp.float32),
            pltpu.SemaphoreType.DMA,
            pltpu.SemaphoreType.DMA,
        ])
    def k(src_h, dst_h, ce_h, ee_h, bd_h, ea_h, eeo_h, nd_h,
          isrc, idst, bd_v, ea_v, ce_v, ee_v, eeo_v, ct_v, nd_sh, sem1, sem2):
        cid = lax.axis_index("c")
        sid = lax.axis_index("s")
        wid = sid * _NC + cid

        @pl.loop(0, _B)
        def _(r):
            for j in range(8):
                ct_v[r, pl.ds(16 * j, 16)] = jnp.zeros((16,), jnp.float32)

        @pl.loop(0, pl.cdiv(_NCH, _NS))
        def _(t):
            c = sid + t * _NS

            @pl.when(c < _NCH)
            def _():
                pltpu.sync_copy(ct_v, nd_sh.at[pl.ds(c * _B, _B)])

        plsc.subcore_barrier()

        @pl.loop(0, _TPW)
        def _(t):
            eb = wid * _EPW + t * _B
            pltpu.sync_copy(src_h.at[pl.ds(eb, _B)], isrc)
            pltpu.sync_copy(dst_h.at[pl.ds(eb, _B)], idst)
            g1 = pltpu.async_copy(bd_h.at[isrc], bd_v, sem1)
            g2 = pltpu.async_copy(ea_h.at[idst], ea_v, sem2)
            pltpu.sync_copy(ce_h.at[pl.ds(eb, _B)], ce_v)
            pltpu.sync_copy(ee_h.at[pl.ds(eb, _B)], ee_v)
            g1.wait()
            g2.wait()

            @pl.loop(0, _B)
            def _(r):
                for j in range(4):
                    sl = pl.ds(16 * j, 16)
                    sh = pl.ds(D + 16 * j, 16)
                    ehat = ce_v[r, sl] + bd_v[r, sh] + ea_v[r, sl]
                    sg = 1.0 / (1.0 + jnp.exp(-ehat))
                    ct_v[r, sl] = sg * bd_v[r, sl]
                    ct_v[r, sh] = sg
                    eeo_v[r, sl] = ee_v[r, sl] + jnp.maximum(ehat, 0.0)

            pltpu.sync_copy(eeo_v, eeo_h.at[pl.ds(eb, _B)])
            pltpu.sync_copy(ct_v, nd_sh.at[idst], add=True)

        plsc.subcore_barrier()

        @pl.loop(0, pl.cdiv(_NCH, _NS))
        def _(t):
            c = sid + t * _NS

            @pl.when(c < _NCH)
            def _():
                pltpu.sync_copy(nd_sh.at[pl.ds(c * _B, _B)],
                                nd_h.at[cid, pl.ds(c * _B, _B)])

    return k(src, dst, ce, ee, bd, ea)


def _sc_final(src, dst, pe, hahb):
    @functools.partial(
        pl.kernel,
        out_type=jax.ShapeDtypeStruct((E, D), jnp.float32),
        mesh=_sc_mesh(),
        scratch_types=[
            pltpu.VMEM((_B,), jnp.int32),
            pltpu.VMEM((_B,), jnp.int32),
            pltpu.VMEM((_B, 2 * D), jnp.float32),
            pltpu.VMEM((_B, 2 * D), jnp.float32),
            pltpu.VMEM((_B, D), jnp.float32),
            pltpu.VMEM((_B, D), jnp.float32),
            pltpu.SemaphoreType.DMA,
            pltpu.SemaphoreType.DMA,
        ])
    def k(src_h, dst_h, pe_h, hahb_h, q_h,
          isrc, idst, ha_v, hb_v, pe_v, q_v, sem1, sem2):
        cid = lax.axis_index("c")
        sid = lax.axis_index("s")
        wid = sid * _NC + cid

        @pl.loop(0, _TPW)
        def _(t):
            eb = wid * _EPW + t * _B
            pltpu.sync_copy(src_h.at[pl.ds(eb, _B)], isrc)
            pltpu.sync_copy(dst_h.at[pl.ds(eb, _B)], idst)
            g1 = pltpu.async_copy(hahb_h.at[isrc], ha_v, sem1)
            g2 = pltpu.async_copy(hahb_h.at[idst], hb_v, sem2)
            pltpu.sync_copy(pe_h.at[pl.ds(eb, _B)], pe_v)
            g1.wait()
            g2.wait()

            @pl.loop(0, _B)
            def _(r):
                for j in range(4):
                    sl = pl.ds(16 * j, 16)
                    sh = pl.ds(D + 16 * j, 16)
                    q_v[r, sl] = jnp.maximum(
                        pe_v[r, sl] + ha_v[r, sl] + hb_v[r, sh], 0.0)

            pltpu.sync_copy(q_v, q_h.at[pl.ds(eb, _B)])

    return k(src, dst, pe, hahb)


def kernel(edge_index, x, e, params):
    src = edge_index[0]
    dst = edge_index[1]
    p = params

    h = _mlp2(x, p['enc_W1'], p['enc_b1'], p['enc_W2'], p['enc_b2'], 1000)
    ee = _mlp2(e, p['e1_W'], p['e1_b'], p['e2_W'], p['e2_b'], 2000)

    for layer in p['gnn']:
        wcat = jnp.concatenate(
            [layer['B_W'], layer['D_W'], layer['E_W'], layer['A_W']], axis=1)
        bcat = jnp.concatenate(
            [layer['B_b'], layer['D_b'], layer['E_b'], layer['A_b']])
        bd, ea = _node_mm2(h, wcat, bcat, 1000)
        ce = _matmul_bias(ee, layer['C_W'], layer['C_b'], 2000)
        ee, ndpart = _sc_layer(src, dst, ce, ee, bd, ea)
        h = _node_update(h, ea, ndpart, 1000)

    w1 = p['sp_W1']
    hahb = _matmul_bias(
        h, jnp.concatenate([w1[:D], w1[D:2 * D]], axis=1),
        jnp.zeros((2 * D,), jnp.float32), 1000)
    pe = _matmul_bias(ee, w1[2 * D:], p['sp_b1'], 2000)
    q = _sc_final(src, dst, pe, hahb)
    scores = _matmul_bias(q, p['sp_W2'], p['sp_b2'], 2000)
    return scores


# speedup vs baseline: 202.6447x; 1.4520x over previous
import functools

import jax
import jax.numpy as jnp
from jax import lax
from jax.experimental import pallas as pl
from jax.experimental.pallas import tpu as pltpu
from jax.experimental.pallas import tpu_sc as plsc

N = 10000
E = 320000
D = 64

_INTERPRET = False


def _mlp2_body(x_ref, w1_ref, b1_ref, w2_ref, b2_ref, o_ref):
    h = jnp.maximum(
        jnp.dot(x_ref[...], w1_ref[...], preferred_element_type=jnp.float32)
        + b1_ref[...], 0.0)
    o_ref[...] = (
        jnp.dot(h, w2_ref[...], preferred_element_type=jnp.float32)
        + b2_ref[...])


def _mlp2(x, w1, b1, w2, b2, blk):
    rows = x.shape[0]
    f_in, f_mid = w1.shape
    f_out = w2.shape[1]
    grid = rows // blk
    return pl.pallas_call(
        _mlp2_body,
        grid=(grid,),
        in_specs=[
            pl.BlockSpec((blk, f_in), lambda i: (i, 0)),
            pl.BlockSpec((f_in, f_mid), lambda i: (0, 0)),
            pl.BlockSpec((1, f_mid), lambda i: (0, 0)),
            pl.BlockSpec((f_mid, f_out), lambda i: (0, 0)),
            pl.BlockSpec((1, f_out), lambda i: (0, 0)),
        ],
        out_specs=pl.BlockSpec((blk, f_out), lambda i: (i, 0)),
        out_shape=jax.ShapeDtypeStruct((rows, f_out), jnp.float32),
        interpret=_INTERPRET,
    )(x, w1, b1.reshape(1, -1), w2, b2.reshape(1, -1))


def _matmul_bias_body(x_ref, w_ref, b_ref, o_ref):
    o_ref[...] = (
        jnp.dot(x_ref[...], w_ref[...], preferred_element_type=jnp.float32)
        + b_ref[...])


def _matmul_bias(x, w, b, blk):
    rows = x.shape[0]
    f_in, f_out = w.shape
    grid = rows // blk
    return pl.pallas_call(
        _matmul_bias_body,
        grid=(grid,),
        in_specs=[
            pl.BlockSpec((blk, f_in), lambda i: (i, 0)),
            pl.BlockSpec((f_in, f_out), lambda i: (0, 0)),
            pl.BlockSpec((1, f_out), lambda i: (0, 0)),
        ],
        out_specs=pl.BlockSpec((blk, f_out), lambda i: (i, 0)),
        out_shape=jax.ShapeDtypeStruct((rows, f_out), jnp.float32),
        interpret=_INTERPRET,
    )(x, w, b.reshape(1, -1))


def _node_mm2_body(x_ref, w_ref, b_ref, o1_ref, o2_ref):
    o = (jnp.dot(x_ref[...], w_ref[...], preferred_element_type=jnp.float32)
         + b_ref[...])
    o1_ref[...] = o[:, :2 * D]
    o2_ref[...] = o[:, 2 * D:]


def _node_mm2(x, w, b, blk):
    grid = N // blk
    return pl.pallas_call(
        _node_mm2_body,
        grid=(grid,),
        in_specs=[
            pl.BlockSpec((blk, D), lambda i: (i, 0)),
            pl.BlockSpec((D, 4 * D), lambda i: (0, 0)),
            pl.BlockSpec((1, 4 * D), lambda i: (0, 0)),
        ],
        out_specs=[
            pl.BlockSpec((blk, 2 * D), lambda i: (i, 0)),
            pl.BlockSpec((blk, 2 * D), lambda i: (i, 0)),
        ],
        out_shape=[
            jax.ShapeDtypeStruct((N, 2 * D), jnp.float32),
            jax.ShapeDtypeStruct((N, 2 * D), jnp.float32),
        ],
        interpret=_INTERPRET,
    )(x, w, b.reshape(1, -1))


def _node_update_body(h_ref, ea_ref, nd_ref, o_ref):
    num = nd_ref[0, :, :D] + nd_ref[1, :, :D]
    den = nd_ref[0, :, D:] + nd_ref[1, :, D:] + 1e-6
    o_ref[...] = h_ref[...] + jnp.maximum(
        ea_ref[:, D:] + num / den, 0.0)


def _node_update(h, ea, ndpart, blk):
    grid = N // blk
    return pl.pallas_call(
        _node_update_body,
        grid=(grid,),
        in_specs=[
            pl.BlockSpec((blk, D), lambda i: (i, 0)),
            pl.BlockSpec((blk, 2 * D), lambda i: (i, 0)),
            pl.BlockSpec((2, blk, 2 * D), lambda i: (0, i, 0)),
        ],
        out_specs=pl.BlockSpec((blk, D), lambda i: (i, 0)),
        out_shape=jax.ShapeDtypeStruct((N, D), jnp.float32),
        interpret=_INTERPRET,
    )(h, ea, ndpart)


_NC = 2
_NS = 16
_NW = _NC * _NS
_B = 40
_EPW = E // _NW
_TPW = _EPW // _B
_NPH = 5
_CPP = _TPW // _NPH
_NCH = N // _B


def _sc_mesh():
    return plsc.VectorSubcoreMesh(
        core_axis_name="c", subcore_axis_name="s",
        num_cores=_NC, num_subcores=_NS)


def _sc_layer(src, dst, ce, ee, bd, ea):
    @functools.partial(
        pl.kernel,
        out_type=[jax.ShapeDtypeStruct((E, D), jnp.float32),
                  jax.ShapeDtypeStruct((_NC, N, 2 * D), jnp.float32)],
        mesh=_sc_mesh(),
        scratch_types=[
            pltpu.VMEM((_CPP * _B,), jnp.int32),
            pltpu.VMEM((_CPP * _B,), jnp.int32),
            pltpu.VMEM((_B,), jnp.int32),
            pltpu.VMEM((_B,), jnp.int32),
            pltpu.VMEM((_B, 2 * D), jnp.float32),
            pltpu.VMEM((_B, 2 * D), jnp.float32),
            pltpu.VMEM((_B, 2 * D), jnp.float32),
            pltpu.VMEM((_B, 2 * D), jnp.float32),
            pltpu.VMEM((_B, D), jnp.float32),
            pltpu.VMEM((_B, D), jnp.float32),
            pltpu.VMEM((_B, D), jnp.float32),
            pltpu.VMEM((_B, D), jnp.float32),
            pltpu.VMEM_SHARED((N, 2 * D), jnp.float32),
            pltpu.SemaphoreType.DMA,
            pltpu.SemaphoreType.DMA,
            pltpu.SemaphoreType.DMA,
            pltpu.SemaphoreType.DMA,
            pltpu.SemaphoreType.DMA,
            pltpu.SemaphoreType.DMA,
        ])
    def k(src_h, dst_h, ce_h, ee_h, bd_h, ea_h, eeo_h, nd_h,
          src_v, dst_v, idst0, idst1, bd0, bd1, ea0, ea1,
          ce0, ce1, ee0, ee1, nd_sh, ins0, ins1, gs0, gs1, outs0, outs1):
        cid = lax.axis_index("c")
        sid = lax.axis_index("s")
        wid = sid * _NC + cid
        IDST = [idst0, idst1]
        BD = [bd0, bd1]
        EA = [ea0, ea1]
        CE = [ce0, ce1]
        EE = [ee0, ee1]
        INS = [ins0, ins1]
        GS = [gs0, gs1]
        OUTS = [outs0, outs1]
        ebase = wid * _EPW

        @pl.loop(0, _B)
        def _(r):
            for j in range(8):
                bd0[r, pl.ds(16 * j, 16)] = jnp.zeros((16,), jnp.float32)

        @pl.loop(0, pl.cdiv(_NCH, _NS))
        def _(t):
            c = sid + t * _NS

            @pl.when(c < _NCH)
            def _():
                pltpu.sync_copy(bd0, nd_sh.at[pl.ds(c * _B, _B)])

        plsc.subcore_barrier()

        def issue_in(g0, c, s):
            eb = ebase + (g0 + c) * _B
            return [
                pltpu.async_copy(ce_h.at[pl.ds(eb, _B)], CE[s], INS[s]),
                pltpu.async_copy(ee_h.at[pl.ds(eb, _B)], EE[s], INS[s]),
                pltpu.async_copy(dst_h.at[pl.ds(eb, _B)], IDST[s], INS[s]),
                pltpu.async_copy(bd_h.at[src_v.at[pl.ds(c * _B, _B)]],
                                 BD[s], GS[s]),
                pltpu.async_copy(ea_h.at[dst_v.at[pl.ds(c * _B, _B)]],
                                 EA[s], GS[s]),
            ]

        def issue_out(g0, c, s):
            eb = ebase + (g0 + c) * _B
            d = pltpu.async_copy(EE[s], eeo_h.at[pl.ds(eb, _B)], OUTS[s])
            pltpu.sync_copy(BD[s], nd_sh.at[IDST[s]], add=True)
            return d

        def compute(s):
            @pl.loop(0, _B)
            def _(r):
                for j in range(4):
                    sl = pl.ds(16 * j, 16)
                    sh = pl.ds(D + 16 * j, 16)
                    bv = BD[s][r, sl]
                    ehat = CE[s][r, sl] + BD[s][r, sh] + EA[s][r, sl]
                    sg = 1.0 / (1.0 + jnp.exp(-ehat))
                    BD[s][r, sl] = sg * bv
                    BD[s][r, sh] = sg
                    EE[s][r, sl] = EE[s][r, sl] + jnp.maximum(ehat, 0.0)

        @pl.loop(0, _NPH)
        def _(ph):
            g0 = ph * _CPP
            eb0 = ebase + g0 * _B
            pltpu.sync_copy(src_h.at[pl.ds(eb0, _CPP * _B)], src_v)
            pltpu.sync_copy(dst_h.at[pl.ds(eb0, _CPP * _B)], dst_v)

            @pl.loop(0, _CPP, step=2)
            def _(t):
                din0 = issue_in(g0, t, 0)
                din1 = issue_in(g0, t + 1, 1)
                for d in din0:
                    d.wait()
                compute(0)
                dout0 = issue_out(g0, t, 0)
                for d in din1:
                    d.wait()
                compute(1)
                dout1 = issue_out(g0, t + 1, 1)
                dout0.wait()
                dout1.wait()

        plsc.subcore_barrier()

        @pl.loop(0, pl.cdiv(_NCH, _NS))
        def _(t):
            c = sid + t * _NS

            @pl.when(c < _NCH)
            def _():
                pltpu.sync_copy(nd_sh.at[pl.ds(c * _B, _B)],
                                nd_h.at[cid, pl.ds(c * _B, _B)])

    return k(src, dst, ce, ee, bd, ea)


def _sc_final(src, dst, pe, hahb):
    @functools.partial(
        pl.kernel,
        out_type=jax.ShapeDtypeStruct((E, D), jnp.float32),
        mesh=_sc_mesh(),
        scratch_types=[
            pltpu.VMEM((_CPP * _B,), jnp.int32),
            pltpu.VMEM((_CPP * _B,), jnp.int32),
            pltpu.VMEM((_B, 2 * D), jnp.float32),
            pltpu.VMEM((_B, 2 * D), jnp.float32),
            pltpu.VMEM((_B, 2 * D), jnp.float32),
            pltpu.VMEM((_B, 2 * D), jnp.float32),
            pltpu.VMEM((_B, D), jnp.float32),
            pltpu.VMEM((_B, D), jnp.float32),
            pltpu.SemaphoreType.DMA,
            pltpu.SemaphoreType.DMA,
            pltpu.SemaphoreType.DMA,
            pltpu.SemaphoreType.DMA,
            pltpu.SemaphoreType.DMA,
            pltpu.SemaphoreType.DMA,
        ])
    def k(src_h, dst_h, pe_h, hahb_h, q_h,
          src_v, dst_v, ha0, ha1, hb0, hb1, pe0, pe1,
          ins0, ins1, gs0, gs1, outs0, outs1):
        cid = lax.axis_index("c")
        sid = lax.axis_index("s")
        wid = sid * _NC + cid
        HA = [ha0, ha1]
        HB = [hb0, hb1]
        PE = [pe0, pe1]
        INS = [ins0, ins1]
        GS = [gs0, gs1]
        OUTS = [outs0, outs1]
        ebase = wid * _EPW

        def issue_in(g0, c, s):
            eb = ebase + (g0 + c) * _B
            return [
                pltpu.async_copy(pe_h.at[pl.ds(eb, _B)], PE[s], INS[s]),
                pltpu.async_copy(hahb_h.at[src_v.at[pl.ds(c * _B, _B)]],
                                 HA[s], GS[s]),
                pltpu.async_copy(hahb_h.at[dst_v.at[pl.ds(c * _B, _B)]],
                                 HB[s], GS[s]),
            ]

        def issue_out(g0, c, s):
            eb = ebase + (g0 + c) * _B
            return pltpu.async_copy(PE[s], q_h.at[pl.ds(eb, _B)], OUTS[s])

        def compute(s):
            @pl.loop(0, _B)
            def _(r):
                for j in range(4):
                    sl = pl.ds(16 * j, 16)
                    sh = pl.ds(D + 16 * j, 16)
                    PE[s][r, sl] = jnp.maximum(
                        PE[s][r, sl] + HA[s][r, sl] + HB[s][r, sh], 0.0)

        @pl.loop(0, _NPH)
        def _(ph):
            g0 = ph * _CPP
            eb0 = ebase + g0 * _B
            pltpu.sync_copy(src_h.at[pl.ds(eb0, _CPP * _B)], src_v)
            pltpu.sync_copy(dst_h.at[pl.ds(eb0, _CPP * _B)], dst_v)

            @pl.loop(0, _CPP, step=2)
            def _(t):
                din0 = issue_in(g0, t, 0)
                din1 = issue_in(g0, t + 1, 1)
                for d in din0:
                    d.wait()
                compute(0)
                dout0 = issue_out(g0, t, 0)
                for d in din1:
                    d.wait()
                compute(1)
                dout1 = issue_out(g0, t + 1, 1)
                dout0.wait()
                dout1.wait()

    return k(src, dst, pe, hahb)


def kernel(edge_index, x, e, params):
    src = edge_index[0]
    dst = edge_index[1]
    p = params

    h = _mlp2(x, p['enc_W1'], p['enc_b1'], p['enc_W2'], p['enc_b2'], 1000)
    ee = _mlp2(e, p['e1_W'], p['e1_b'], p['e2_W'], p['e2_b'], 2000)

    for layer in p['gnn']:
        wcat = jnp.concatenate(
            [layer['B_W'], layer['D_W'], layer['E_W'], layer['A_W']], axis=1)
        bcat = jnp.concatenate(
            [layer['B_b'], layer['D_b'], layer['E_b'], layer['A_b']])
        bd, ea = _node_mm2(h, wcat, bcat, 1000)
        ce = _matmul_bias(ee, layer['C_W'], layer['C_b'], 2000)
        ee, ndpart = _sc_layer(src, dst, ce, ee, bd, ea)
        h = _node_update(h, ea, ndpart, 1000)

    w1 = p['sp_W1']
    hahb = _matmul_bias(
        h, jnp.concatenate([w1[:D], w1[D:2 * D]], axis=1),
        jnp.zeros((2 * D,), jnp.float32), 1000)
    pe = _matmul_bias(ee, w1[2 * D:], p['sp_b1'], 2000)
    q = _sc_final(src, dst, pe, hahb)
    scores = _matmul_bias(q, p['sp_W2'], p['sp_b2'], 2000)
    return scores


# speedup vs baseline: 226.9726x; 1.1201x over previous
import functools

import jax
import jax.numpy as jnp
from jax import lax
from jax.experimental import pallas as pl
from jax.experimental.pallas import tpu as pltpu
from jax.experimental.pallas import tpu_sc as plsc

N = 10000
E = 320000
D = 64

_INTERPRET = False


def _mlp2_body(x_ref, w1_ref, b1_ref, w2_ref, b2_ref, o_ref):
    h = jnp.maximum(
        jnp.dot(x_ref[...], w1_ref[...], preferred_element_type=jnp.float32)
        + b1_ref[...], 0.0)
    o_ref[...] = (
        jnp.dot(h, w2_ref[...], preferred_element_type=jnp.float32)
        + b2_ref[...])


def _mlp2(x, w1, b1, w2, b2, blk):
    rows = x.shape[0]
    f_in, f_mid = w1.shape
    f_out = w2.shape[1]
    grid = rows // blk
    return pl.pallas_call(
        _mlp2_body,
        grid=(grid,),
        in_specs=[
            pl.BlockSpec((blk, f_in), lambda i: (i, 0)),
            pl.BlockSpec((f_in, f_mid), lambda i: (0, 0)),
            pl.BlockSpec((1, f_mid), lambda i: (0, 0)),
            pl.BlockSpec((f_mid, f_out), lambda i: (0, 0)),
            pl.BlockSpec((1, f_out), lambda i: (0, 0)),
        ],
        out_specs=pl.BlockSpec((blk, f_out), lambda i: (i, 0)),
        out_shape=jax.ShapeDtypeStruct((rows, f_out), jnp.float32),
        interpret=_INTERPRET,
    )(x, w1, b1.reshape(1, -1), w2, b2.reshape(1, -1))


def _matmul_bias_body(x_ref, w_ref, b_ref, o_ref):
    o_ref[...] = (
        jnp.dot(x_ref[...], w_ref[...], preferred_element_type=jnp.float32)
        + b_ref[...])


def _matmul_bias(x, w, b, blk):
    rows = x.shape[0]
    f_in, f_out = w.shape
    grid = rows // blk
    return pl.pallas_call(
        _matmul_bias_body,
        grid=(grid,),
        in_specs=[
            pl.BlockSpec((blk, f_in), lambda i: (i, 0)),
            pl.BlockSpec((f_in, f_out), lambda i: (0, 0)),
            pl.BlockSpec((1, f_out), lambda i: (0, 0)),
        ],
        out_specs=pl.BlockSpec((blk, f_out), lambda i: (i, 0)),
        out_shape=jax.ShapeDtypeStruct((rows, f_out), jnp.float32),
        interpret=_INTERPRET,
    )(x, w, b.reshape(1, -1))


def _node_mm2_body(x_ref, w_ref, b_ref, o1_ref, o2_ref):
    o = (jnp.dot(x_ref[...], w_ref[...], preferred_element_type=jnp.float32)
         + b_ref[...])
    o1_ref[...] = o[:, :2 * D]
    o2_ref[...] = o[:, 2 * D:]


def _node_mm2(x, w, b, blk):
    grid = N // blk
    return pl.pallas_call(
        _node_mm2_body,
        grid=(grid,),
        in_specs=[
            pl.BlockSpec((blk, D), lambda i: (i, 0)),
            pl.BlockSpec((D, 4 * D), lambda i: (0, 0)),
            pl.BlockSpec((1, 4 * D), lambda i: (0, 0)),
        ],
        out_specs=[
            pl.BlockSpec((blk, 2 * D), lambda i: (i, 0)),
            pl.BlockSpec((blk, 2 * D), lambda i: (i, 0)),
        ],
        out_shape=[
            jax.ShapeDtypeStruct((N, 2 * D), jnp.float32),
            jax.ShapeDtypeStruct((N, 2 * D), jnp.float32),
        ],
        interpret=_INTERPRET,
    )(x, w, b.reshape(1, -1))


def _node_update_body(h_ref, ea_ref, nd_ref, o_ref):
    num = nd_ref[0, :, :D] + nd_ref[1, :, :D]
    den = nd_ref[0, :, D:] + nd_ref[1, :, D:] + 1e-6
    o_ref[...] = h_ref[...] + jnp.maximum(
        ea_ref[:, D:] + num / den, 0.0)


def _node_update(h, ea, ndpart, blk):
    grid = N // blk
    return pl.pallas_call(
        _node_update_body,
        grid=(grid,),
        in_specs=[
            pl.BlockSpec((blk, D), lambda i: (i, 0)),
            pl.BlockSpec((blk, 2 * D), lambda i: (i, 0)),
            pl.BlockSpec((2, blk, 2 * D), lambda i: (0, i, 0)),
        ],
        out_specs=pl.BlockSpec((blk, D), lambda i: (i, 0)),
        out_shape=jax.ShapeDtypeStruct((N, D), jnp.float32),
        interpret=_INTERPRET,
    )(h, ea, ndpart)


_NC = 2
_NS = 16
_NW = _NC * _NS
_B = 40
_EPW = E // _NW
_TPW = _EPW // _B
_NPH = 5
_CPP = _TPW // _NPH
_NCH = N // _B


def _sc_mesh():
    return plsc.VectorSubcoreMesh(
        core_axis_name="c", subcore_axis_name="s",
        num_cores=_NC, num_subcores=_NS)


def _sc_layer(src, dst, ce, ee, bd, ea):
    @functools.partial(
        pl.kernel,
        out_type=[jax.ShapeDtypeStruct((E, D), jnp.float32),
                  jax.ShapeDtypeStruct((_NC, N, 2 * D), jnp.float32)],
        mesh=_sc_mesh(),
        scratch_types=[
            pltpu.VMEM((_CPP * _B,), jnp.int32),
            pltpu.VMEM((_CPP * _B,), jnp.int32),
            pltpu.VMEM((_B,), jnp.int32),
            pltpu.VMEM((_B,), jnp.int32),
            pltpu.VMEM((_B, 2 * D), jnp.float32),
            pltpu.VMEM((_B, 2 * D), jnp.float32),
            pltpu.VMEM((_B, 2 * D), jnp.float32),
            pltpu.VMEM((_B, 2 * D), jnp.float32),
            pltpu.VMEM((_B, D), jnp.float32),
            pltpu.VMEM((_B, D), jnp.float32),
            pltpu.VMEM((_B, D), jnp.float32),
            pltpu.VMEM((_B, D), jnp.float32),
            pltpu.VMEM_SHARED((N, 2 * D), jnp.float32),
            pltpu.SemaphoreType.DMA,
            pltpu.SemaphoreType.DMA,
            pltpu.SemaphoreType.DMA,
            pltpu.SemaphoreType.DMA,
            pltpu.SemaphoreType.DMA,
            pltpu.SemaphoreType.DMA,
        ])
    def k(src_h, dst_h, ce_h, ee_h, bd_h, ea_h, eeo_h, nd_h,
          src_v, dst_v, idst0, idst1, bd0, bd1, ea0, ea1,
          ce0, ce1, ee0, ee1, nd_sh, ins0, ins1, gs0, gs1, outs0, outs1):
        cid = lax.axis_index("c")
        sid = lax.axis_index("s")
        wid = sid * _NC + cid
        IDST = [idst0, idst1]
        BD = [bd0, bd1]
        EA = [ea0, ea1]
        CE = [ce0, ce1]
        EE = [ee0, ee1]
        INS = [ins0, ins1]
        GS = [gs0, gs1]
        OUTS = [outs0, outs1]
        ebase = wid * _EPW

        @pl.loop(0, _B)
        def _(r):
            for j in range(8):
                bd0[r, pl.ds(16 * j, 16)] = jnp.zeros((16,), jnp.float32)

        @pl.loop(0, pl.cdiv(_NCH, _NS))
        def _(t):
            c = sid + t * _NS

            @pl.when(c < _NCH)
            def _():
                pltpu.sync_copy(bd0, nd_sh.at[pl.ds(c * _B, _B)])

        plsc.subcore_barrier()

        def issue_in(g0, c, s):
            eb = ebase + (g0 + c) * _B
            return [
                pltpu.async_copy(ce_h.at[pl.ds(eb, _B)], CE[s], INS[s]),
                pltpu.async_copy(ee_h.at[pl.ds(eb, _B)], EE[s], INS[s]),
                pltpu.async_copy(dst_h.at[pl.ds(eb, _B)], IDST[s], INS[s]),
                pltpu.async_copy(bd_h.at[src_v.at[pl.ds(c * _B, _B)]],
                                 BD[s], GS[s]),
                pltpu.async_copy(ea_h.at[dst_v.at[pl.ds(c * _B, _B)]],
                                 EA[s], GS[s]),
            ]

        def issue_out(g0, c, s):
            eb = ebase + (g0 + c) * _B
            return [
                pltpu.async_copy(EE[s], eeo_h.at[pl.ds(eb, _B)], OUTS[s]),
                pltpu.async_copy(BD[s], nd_sh.at[IDST[s]], GS[s], add=True),
            ]

        def compute(s):
            @pl.loop(0, _B, step=2)
            def _(r0):
                for dr in range(2):
                    r = r0 + dr
                    for j in range(4):
                        sl = pl.ds(16 * j, 16)
                        sh = pl.ds(D + 16 * j, 16)
                        bv = BD[s][r, sl]
                        ehat = CE[s][r, sl] + BD[s][r, sh] + EA[s][r, sl]
                        sg = 1.0 / (1.0 + jnp.exp(-ehat))
                        BD[s][r, sl] = sg * bv
                        BD[s][r, sh] = sg
                        EE[s][r, sl] = EE[s][r, sl] + jnp.maximum(ehat, 0.0)

        @pl.loop(0, _NPH)
        def _(ph):
            g0 = ph * _CPP
            eb0 = ebase + g0 * _B
            pltpu.sync_copy(src_h.at[pl.ds(eb0, _CPP * _B)], src_v)
            pltpu.sync_copy(dst_h.at[pl.ds(eb0, _CPP * _B)], dst_v)

            @pl.loop(0, _CPP, step=2)
            def _(t):
                din0 = issue_in(g0, t, 0)
                din1 = issue_in(g0, t + 1, 1)
                for d in din0:
                    d.wait()
                compute(0)
                dout0 = issue_out(g0, t, 0)
                for d in din1:
                    d.wait()
                compute(1)
                dout1 = issue_out(g0, t + 1, 1)
                for d in dout0:
                    d.wait()
                for d in dout1:
                    d.wait()

        plsc.subcore_barrier()

        @pl.loop(0, pl.cdiv(_NCH, _NS))
        def _(t):
            c = sid + t * _NS

            @pl.when(c < _NCH)
            def _():
                pltpu.sync_copy(nd_sh.at[pl.ds(c * _B, _B)],
                                nd_h.at[cid, pl.ds(c * _B, _B)])

    return k(src, dst, ce, ee, bd, ea)


def _sc_final(src, dst, pe, hahb):
    @functools.partial(
        pl.kernel,
        out_type=jax.ShapeDtypeStruct((E, D), jnp.float32),
        mesh=_sc_mesh(),
        scratch_types=[
            pltpu.VMEM((_CPP * _B,), jnp.int32),
            pltpu.VMEM((_CPP * _B,), jnp.int32),
            pltpu.VMEM((_B, 2 * D), jnp.float32),
            pltpu.VMEM((_B, 2 * D), jnp.float32),
            pltpu.VMEM((_B, 2 * D), jnp.float32),
            pltpu.VMEM((_B, 2 * D), jnp.float32),
            pltpu.VMEM((_B, D), jnp.float32),
            pltpu.VMEM((_B, D), jnp.float32),
            pltpu.SemaphoreType.DMA,
            pltpu.SemaphoreType.DMA,
            pltpu.SemaphoreType.DMA,
            pltpu.SemaphoreType.DMA,
            pltpu.SemaphoreType.DMA,
            pltpu.SemaphoreType.DMA,
        ])
    def k(src_h, dst_h, pe_h, hahb_h, q_h,
          src_v, dst_v, ha0, ha1, hb0, hb1, pe0, pe1,
          ins0, ins1, gs0, gs1, outs0, outs1):
        cid = lax.axis_index("c")
        sid = lax.axis_index("s")
        wid = sid * _NC + cid
        HA = [ha0, ha1]
        HB = [hb0, hb1]
        PE = [pe0, pe1]
        INS = [ins0, ins1]
        GS = [gs0, gs1]
        OUTS = [outs0, outs1]
        ebase = wid * _EPW

        def issue_in(g0, c, s):
            eb = ebase + (g0 + c) * _B
            return [
                pltpu.async_copy(pe_h.at[pl.ds(eb, _B)], PE[s], INS[s]),
                pltpu.async_copy(hahb_h.at[src_v.at[pl.ds(c * _B, _B)]],
                                 HA[s], GS[s]),
                pltpu.async_copy(hahb_h.at[dst_v.at[pl.ds(c * _B, _B)]],
                                 HB[s], GS[s]),
            ]

        def issue_out(g0, c, s):
            eb = ebase + (g0 + c) * _B
            return pltpu.async_copy(PE[s], q_h.at[pl.ds(eb, _B)], OUTS[s])

        def compute(s):
            @pl.loop(0, _B, step=2)
            def _(r0):
                for dr in range(2):
                    r = r0 + dr
                    for j in range(4):
                        sl = pl.ds(16 * j, 16)
                        sh = pl.ds(D + 16 * j, 16)
                        PE[s][r, sl] = jnp.maximum(
                            PE[s][r, sl] + HA[s][r, sl] + HB[s][r, sh], 0.0)

        @pl.loop(0, _NPH)
        def _(ph):
            g0 = ph * _CPP
            eb0 = ebase + g0 * _B
            pltpu.sync_copy(src_h.at[pl.ds(eb0, _CPP * _B)], src_v)
            pltpu.sync_copy(dst_h.at[pl.ds(eb0, _CPP * _B)], dst_v)

            @pl.loop(0, _CPP, step=2)
            def _(t):
                din0 = issue_in(g0, t, 0)
                din1 = issue_in(g0, t + 1, 1)
                for d in din0:
                    d.wait()
                compute(0)
                dout0 = issue_out(g0, t, 0)
                for d in din1:
                    d.wait()
                compute(1)
                dout1 = issue_out(g0, t + 1, 1)
                dout0.wait()
                dout1.wait()

    return k(src, dst, pe, hahb)


def kernel(edge_index, x, e, params):
    src = edge_index[0]
    dst = edge_index[1]
    p = params

    h = _mlp2(x, p['enc_W1'], p['enc_b1'], p['enc_W2'], p['enc_b2'], 1000)
    ee = _mlp2(e, p['e1_W'], p['e1_b'], p['e2_W'], p['e2_b'], 2000)

    for layer in p['gnn']:
        wcat = jnp.concatenate(
            [layer['B_W'], layer['D_W'], layer['E_W'], layer['A_W']], axis=1)
        bcat = jnp.concatenate(
            [layer['B_b'], layer['D_b'], layer['E_b'], layer['A_b']])
        bd, ea = _node_mm2(h, wcat, bcat, 1000)
        ce = _matmul_bias(ee, layer['C_W'], layer['C_b'], 2000)
        ee, ndpart = _sc_layer(src, dst, ce, ee, bd, ea)
        h = _node_update(h, ea, ndpart, 1000)

    w1 = p['sp_W1']
    hahb = _matmul_bias(
        h, jnp.concatenate([w1[:D], w1[D:2 * D]], axis=1),
        jnp.zeros((2 * D,), jnp.float32), 1000)
    pe = _matmul_bias(ee, w1[2 * D:], p['sp_b1'], 2000)
    q = _sc_final(src, dst, pe, hahb)
    scores = _matmul_bias(q, p['sp_W2'], p['sp_b2'], 2000)
    return scores


# speedup vs baseline: 238.8898x; 1.0525x over previous
import functools

import jax
import jax.numpy as jnp
from jax import lax
from jax.experimental import pallas as pl
from jax.experimental.pallas import tpu as pltpu
from jax.experimental.pallas import tpu_sc as plsc

N = 10000
E = 320000
D = 64

_INTERPRET = False


def _mlp2_body(x_ref, w1_ref, b1_ref, w2_ref, b2_ref, o_ref):
    h = jnp.maximum(
        jnp.dot(x_ref[...], w1_ref[...], preferred_element_type=jnp.float32)
        + b1_ref[...], 0.0)
    o_ref[...] = (
        jnp.dot(h, w2_ref[...], preferred_element_type=jnp.float32)
        + b2_ref[...])


def _mlp2(x, w1, b1, w2, b2, blk):
    rows = x.shape[0]
    f_in, f_mid = w1.shape
    f_out = w2.shape[1]
    grid = rows // blk
    return pl.pallas_call(
        _mlp2_body,
        grid=(grid,),
        in_specs=[
            pl.BlockSpec((blk, f_in), lambda i: (i, 0)),
            pl.BlockSpec((f_in, f_mid), lambda i: (0, 0)),
            pl.BlockSpec((1, f_mid), lambda i: (0, 0)),
            pl.BlockSpec((f_mid, f_out), lambda i: (0, 0)),
            pl.BlockSpec((1, f_out), lambda i: (0, 0)),
        ],
        out_specs=pl.BlockSpec((blk, f_out), lambda i: (i, 0)),
        out_shape=jax.ShapeDtypeStruct((rows, f_out), jnp.float32),
        interpret=_INTERPRET,
    )(x, w1, b1.reshape(1, -1), w2, b2.reshape(1, -1))


def _matmul_bias_body(x_ref, w_ref, b_ref, o_ref):
    o_ref[...] = (
        jnp.dot(x_ref[...], w_ref[...], preferred_element_type=jnp.float32)
        + b_ref[...])


def _matmul_bias(x, w, b, blk):
    rows = x.shape[0]
    f_in, f_out = w.shape
    grid = rows // blk
    return pl.pallas_call(
        _matmul_bias_body,
        grid=(grid,),
        in_specs=[
            pl.BlockSpec((blk, f_in), lambda i: (i, 0)),
            pl.BlockSpec((f_in, f_out), lambda i: (0, 0)),
            pl.BlockSpec((1, f_out), lambda i: (0, 0)),
        ],
        out_specs=pl.BlockSpec((blk, f_out), lambda i: (i, 0)),
        out_shape=jax.ShapeDtypeStruct((rows, f_out), jnp.float32),
        interpret=_INTERPRET,
    )(x, w, b.reshape(1, -1))


def _node_mm2_body(x_ref, w_ref, b_ref, o1_ref, o2_ref):
    o = (jnp.dot(x_ref[...], w_ref[...], preferred_element_type=jnp.float32)
         + b_ref[...])
    o1_ref[...] = o[:, :2 * D]
    o2_ref[...] = o[:, 2 * D:]


def _node_mm2(x, w, b, blk):
    grid = N // blk
    return pl.pallas_call(
        _node_mm2_body,
        grid=(grid,),
        in_specs=[
            pl.BlockSpec((blk, D), lambda i: (i, 0)),
            pl.BlockSpec((D, 4 * D), lambda i: (0, 0)),
            pl.BlockSpec((1, 4 * D), lambda i: (0, 0)),
        ],
        out_specs=[
            pl.BlockSpec((blk, 2 * D), lambda i: (i, 0)),
            pl.BlockSpec((blk, 2 * D), lambda i: (i, 0)),
        ],
        out_shape=[
            jax.ShapeDtypeStruct((N, 2 * D), jnp.float32),
            jax.ShapeDtypeStruct((N, 2 * D), jnp.float32),
        ],
        interpret=_INTERPRET,
    )(x, w, b.reshape(1, -1))


_NB = 10
_EBLK = 2000
_NBLK = 1000


def _nodeidx(i):
    return (jnp.minimum(i, _NB - 1), 0)


def _upd(h_ref, eap_ref, nd_ref):
    num = nd_ref[0, :, :D] + nd_ref[1, :, :D]
    den = nd_ref[0, :, D:] + nd_ref[1, :, D:] + 1e-6
    return h_ref[...] + jnp.maximum(eap_ref[:, D:] + num / den, 0.0)


def _interlayer_body(h_ref, eap_ref, nd_ref, wcat_ref, bcat_ref,
                     cw_ref, cb_ref, ee_ref,
                     hn_ref, bd_ref, ean_ref, ce_ref):
    i = pl.program_id(0)

    @pl.when(i < _NB)
    def _():
        hn = _upd(h_ref, eap_ref, nd_ref)
        hn_ref[...] = hn
        o = (jnp.dot(hn, wcat_ref[...], preferred_element_type=jnp.float32)
             + bcat_ref[...])
        bd_ref[...] = o[:, :2 * D]
        ean_ref[...] = o[:, 2 * D:]

    ce_ref[...] = (
        jnp.dot(ee_ref[...], cw_ref[...], preferred_element_type=jnp.float32)
        + cb_ref[...])


def _interlayer(h, ea_prev, nd, wcat, bcat, cw, cb, ee):
    return pl.pallas_call(
        _interlayer_body,
        grid=(E // _EBLK,),
        in_specs=[
            pl.BlockSpec((_NBLK, D), _nodeidx),
            pl.BlockSpec((_NBLK, 2 * D), _nodeidx),
            pl.BlockSpec((2, _NBLK, 2 * D), lambda i: (0, *_nodeidx(i))),
            pl.BlockSpec((D, 4 * D), lambda i: (0, 0)),
            pl.BlockSpec((1, 4 * D), lambda i: (0, 0)),
            pl.BlockSpec((D, D), lambda i: (0, 0)),
            pl.BlockSpec((1, D), lambda i: (0, 0)),
            pl.BlockSpec((_EBLK, D), lambda i: (i, 0)),
        ],
        out_specs=[
            pl.BlockSpec((_NBLK, D), _nodeidx),
            pl.BlockSpec((_NBLK, 2 * D), _nodeidx),
            pl.BlockSpec((_NBLK, 2 * D), _nodeidx),
            pl.BlockSpec((_EBLK, D), lambda i: (i, 0)),
        ],
        out_shape=[
            jax.ShapeDtypeStruct((N, D), jnp.float32),
            jax.ShapeDtypeStruct((N, 2 * D), jnp.float32),
            jax.ShapeDtypeStruct((N, 2 * D), jnp.float32),
            jax.ShapeDtypeStruct((E, D), jnp.float32),
        ],
        interpret=_INTERPRET,
    )(h, ea_prev, nd, wcat, bcat.reshape(1, -1), cw, cb.reshape(1, -1), ee)


def _final_prep_body(h_ref, eap_ref, nd_ref, w12_ref, w1c_ref, b1_ref,
                     ee_ref, hahb_ref, pe_ref):
    i = pl.program_id(0)

    @pl.when(i < _NB)
    def _():
        hn = _upd(h_ref, eap_ref, nd_ref)
        hahb_ref[...] = jnp.dot(hn, w12_ref[...],
                                preferred_element_type=jnp.float32)

    pe_ref[...] = (
        jnp.dot(ee_ref[...], w1c_ref[...], preferred_element_type=jnp.float32)
        + b1_ref[...])


def _final_prep(h, ea_prev, nd, w12, w1c, b1, ee):
    return pl.pallas_call(
        _final_prep_body,
        grid=(E // _EBLK,),
        in_specs=[
            pl.BlockSpec((_NBLK, D), _nodeidx),
            pl.BlockSpec((_NBLK, 2 * D), _nodeidx),
            pl.BlockSpec((2, _NBLK, 2 * D), lambda i: (0, *_nodeidx(i))),
            pl.BlockSpec((D, 2 * D), lambda i: (0, 0)),
            pl.BlockSpec((D, D), lambda i: (0, 0)),
            pl.BlockSpec((1, D), lambda i: (0, 0)),
            pl.BlockSpec((_EBLK, D), lambda i: (i, 0)),
        ],
        out_specs=[
            pl.BlockSpec((_NBLK, 2 * D), _nodeidx),
            pl.BlockSpec((_EBLK, D), lambda i: (i, 0)),
        ],
        out_shape=[
            jax.ShapeDtypeStruct((N, 2 * D), jnp.float32),
            jax.ShapeDtypeStruct((E, D), jnp.float32),
        ],
        interpret=_INTERPRET,
    )(h, ea_prev, nd, w12, w1c, b1.reshape(1, -1), ee)


def _encoders_body(x_ref, nw1_ref, nb1_ref, nw2_ref, nb2_ref,
                   wcat_ref, bcat_ref,
                   e_ref, ew1_ref, eb1_ref, ew2_ref, eb2_ref,
                   cw_ref, cb_ref,
                   h_ref, bd_ref, ea_ref, ee_ref, ce_ref):
    i = pl.program_id(0)

    @pl.when(i < _NB)
    def _():
        t = jnp.maximum(
            jnp.dot(x_ref[...], nw1_ref[...],
                    preferred_element_type=jnp.float32) + nb1_ref[...], 0.0)
        h = (jnp.dot(t, nw2_ref[...], preferred_element_type=jnp.float32)
             + nb2_ref[...])
        h_ref[...] = h
        o = (jnp.dot(h, wcat_ref[...], preferred_element_type=jnp.float32)
             + bcat_ref[...])
        bd_ref[...] = o[:, :2 * D]
        ea_ref[...] = o[:, 2 * D:]

    t = jnp.maximum(
        jnp.dot(e_ref[...], ew1_ref[...],
                preferred_element_type=jnp.float32) + eb1_ref[...], 0.0)
    ee = (jnp.dot(t, ew2_ref[...], preferred_element_type=jnp.float32)
          + eb2_ref[...])
    ee_ref[...] = ee
    ce_ref[...] = (
        jnp.dot(ee, cw_ref[...], preferred_element_type=jnp.float32)
        + cb_ref[...])


def _encoders(x, nw1, nb1, nw2, nb2, wcat, bcat,
              e, ew1, eb1, ew2, eb2, cw, cb):
    hid = ew1.shape[1]
    return pl.pallas_call(
        _encoders_body,
        grid=(E // _EBLK,),
        in_specs=[
            pl.BlockSpec((_NBLK, x.shape[1]), _nodeidx),
            pl.BlockSpec((x.shape[1], D), lambda i: (0, 0)),
            pl.BlockSpec((1, D), lambda i: (0, 0)),
            pl.BlockSpec((D, D), lambda i: (0, 0)),
            pl.BlockSpec((1, D), lambda i: (0, 0)),
            pl.BlockSpec((D, 4 * D), lambda i: (0, 0)),
            pl.BlockSpec((1, 4 * D), lambda i: (0, 0)),
            pl.BlockSpec((_EBLK, e.shape[1]), lambda i: (i, 0)),
            pl.BlockSpec((e.shape[1], hid), lambda i: (0, 0)),
            pl.BlockSpec((1, hid), lambda i: (0, 0)),
            pl.BlockSpec((hid, D), lambda i: (0, 0)),
            pl.BlockSpec((1, D), lambda i: (0, 0)),
            pl.BlockSpec((D, D), lambda i: (0, 0)),
            pl.BlockSpec((1, D), lambda i: (0, 0)),
        ],
        out_specs=[
            pl.BlockSpec((_NBLK, D), _nodeidx),
            pl.BlockSpec((_NBLK, 2 * D), _nodeidx),
            pl.BlockSpec((_NBLK, 2 * D), _nodeidx),
            pl.BlockSpec((_EBLK, D), lambda i: (i, 0)),
            pl.BlockSpec((_EBLK, D), lambda i: (i, 0)),
        ],
        out_shape=[
            jax.ShapeDtypeStruct((N, D), jnp.float32),
            jax.ShapeDtypeStruct((N, 2 * D), jnp.float32),
            jax.ShapeDtypeStruct((N, 2 * D), jnp.float32),
            jax.ShapeDtypeStruct((E, D), jnp.float32),
            jax.ShapeDtypeStruct((E, D), jnp.float32),
        ],
        interpret=_INTERPRET,
    )(x, nw1, nb1.reshape(1, -1), nw2, nb2.reshape(1, -1),
      wcat, bcat.reshape(1, -1),
      e, ew1, eb1.reshape(1, -1), ew2, eb2.reshape(1, -1),
      cw, cb.reshape(1, -1))


_NC = 2
_NS = 16
_NW = _NC * _NS
_B = 40
_EPW = E // _NW
_TPW = _EPW // _B
_NPH = 5
_CPP = _TPW // _NPH
_NCH = N // _B


def _sc_mesh():
    return plsc.VectorSubcoreMesh(
        core_axis_name="c", subcore_axis_name="s",
        num_cores=_NC, num_subcores=_NS)


def _sc_layer(src, dst, ce, ee, bd, ea):
    @functools.partial(
        pl.kernel,
        out_type=[jax.ShapeDtypeStruct((E, D), jnp.float32),
                  jax.ShapeDtypeStruct((_NC, N, 2 * D), jnp.float32)],
        mesh=_sc_mesh(),
        scratch_types=[
            pltpu.VMEM((_CPP * _B,), jnp.int32),
            pltpu.VMEM((_CPP * _B,), jnp.int32),
            pltpu.VMEM((_B,), jnp.int32),
            pltpu.VMEM((_B,), jnp.int32),
            pltpu.VMEM((_B, 2 * D), jnp.float32),
            pltpu.VMEM((_B, 2 * D), jnp.float32),
            pltpu.VMEM((_B, 2 * D), jnp.float32),
            pltpu.VMEM((_B, 2 * D), jnp.float32),
            pltpu.VMEM((_B, D), jnp.float32),
            pltpu.VMEM((_B, D), jnp.float32),
            pltpu.VMEM((_B, D), jnp.float32),
            pltpu.VMEM((_B, D), jnp.float32),
            pltpu.VMEM_SHARED((N, 2 * D), jnp.float32),
            pltpu.SemaphoreType.DMA,
            pltpu.SemaphoreType.DMA,
            pltpu.SemaphoreType.DMA,
            pltpu.SemaphoreType.DMA,
            pltpu.SemaphoreType.DMA,
            pltpu.SemaphoreType.DMA,
        ])
    def k(src_h, dst_h, ce_h, ee_h, bd_h, ea_h, eeo_h, nd_h,
          src_v, dst_v, idst0, idst1, bd0, bd1, ea0, ea1,
          ce0, ce1, ee0, ee1, nd_sh, ins0, ins1, gs0, gs1, outs0, outs1):
        cid = lax.axis_index("c")
        sid = lax.axis_index("s")
        wid = sid * _NC + cid
        IDST = [idst0, idst1]
        BD = [bd0, bd1]
        EA = [ea0, ea1]
        CE = [ce0, ce1]
        EE = [ee0, ee1]
        INS = [ins0, ins1]
        GS = [gs0, gs1]
        OUTS = [outs0, outs1]
        ebase = wid * _EPW

        @pl.loop(0, _B)
        def _(r):
            for j in range(8):
                bd0[r, pl.ds(16 * j, 16)] = jnp.zeros((16,), jnp.float32)

        @pl.loop(0, pl.cdiv(_NCH, _NS))
        def _(t):
            c = sid + t * _NS

            @pl.when(c < _NCH)
            def _():
                pltpu.sync_copy(bd0, nd_sh.at[pl.ds(c * _B, _B)])

        plsc.subcore_barrier()

        def issue_in(g0, c, s):
            eb = ebase + (g0 + c) * _B
            return [
                pltpu.async_copy(ce_h.at[pl.ds(eb, _B)], CE[s], INS[s]),
                pltpu.async_copy(ee_h.at[pl.ds(eb, _B)], EE[s], INS[s]),
                pltpu.async_copy(dst_h.at[pl.ds(eb, _B)], IDST[s], INS[s]),
                pltpu.async_copy(bd_h.at[src_v.at[pl.ds(c * _B, _B)]],
                                 BD[s], GS[s]),
                pltpu.async_copy(ea_h.at[dst_v.at[pl.ds(c * _B, _B)]],
                                 EA[s], GS[s]),
            ]

        def issue_out(g0, c, s):
            eb = ebase + (g0 + c) * _B
            return [
                pltpu.async_copy(EE[s], eeo_h.at[pl.ds(eb, _B)], OUTS[s]),
                pltpu.async_copy(BD[s], nd_sh.at[IDST[s]], GS[s], add=True),
            ]

        def compute(s):
            @pl.loop(0, _B, step=2)
            def _(r0):
                for dr in range(2):
                    r = r0 + dr
                    for j in range(4):
                        sl = pl.ds(16 * j, 16)
                        sh = pl.ds(D + 16 * j, 16)
                        bv = BD[s][r, sl]
                        ehat = CE[s][r, sl] + BD[s][r, sh] + EA[s][r, sl]
                        sg = 1.0 / (1.0 + jnp.exp(-ehat))
                        BD[s][r, sl] = sg * bv
                        BD[s][r, sh] = sg
                        EE[s][r, sl] = EE[s][r, sl] + jnp.maximum(ehat, 0.0)

        @pl.loop(0, _NPH)
        def _(ph):
            g0 = ph * _CPP
            eb0 = ebase + g0 * _B
            pltpu.sync_copy(src_h.at[pl.ds(eb0, _CPP * _B)], src_v)
            pltpu.sync_copy(dst_h.at[pl.ds(eb0, _CPP * _B)], dst_v)

            @pl.loop(0, _CPP, step=2)
            def _(t):
                din0 = issue_in(g0, t, 0)
                din1 = issue_in(g0, t + 1, 1)
                for d in din0:
                    d.wait()
                compute(0)
                dout0 = issue_out(g0, t, 0)
                for d in din1:
                    d.wait()
                compute(1)
                dout1 = issue_out(g0, t + 1, 1)
                for d in dout0:
                    d.wait()
                for d in dout1:
                    d.wait()

        plsc.subcore_barrier()

        @pl.loop(0, pl.cdiv(_NCH, _NS))
        def _(t):
            c = sid + t * _NS

            @pl.when(c < _NCH)
            def _():
                pltpu.sync_copy(nd_sh.at[pl.ds(c * _B, _B)],
                                nd_h.at[cid, pl.ds(c * _B, _B)])

    return k(src, dst, ce, ee, bd, ea)


def _sc_final(src, dst, pe, hahb):
    @functools.partial(
        pl.kernel,
        out_type=jax.ShapeDtypeStruct((E, D), jnp.float32),
        mesh=_sc_mesh(),
        scratch_types=[
            pltpu.VMEM((_CPP * _B,), jnp.int32),
            pltpu.VMEM((_CPP * _B,), jnp.int32),
            pltpu.VMEM((_B, 2 * D), jnp.float32),
            pltpu.VMEM((_B, 2 * D), jnp.float32),
            pltpu.VMEM((_B, 2 * D), jnp.float32),
            pltpu.VMEM((_B, 2 * D), jnp.float32),
            pltpu.VMEM((_B, D), jnp.float32),
            pltpu.VMEM((_B, D), jnp.float32),
            pltpu.SemaphoreType.DMA,
            pltpu.SemaphoreType.DMA,
            pltpu.SemaphoreType.DMA,
            pltpu.SemaphoreType.DMA,
            pltpu.SemaphoreType.DMA,
            pltpu.SemaphoreType.DMA,
        ])
    def k(src_h, dst_h, pe_h, hahb_h, q_h,
          src_v, dst_v, ha0, ha1, hb0, hb1, pe0, pe1,
          ins0, ins1, gs0, gs1, outs0, outs1):
        cid = lax.axis_index("c")
        sid = lax.axis_index("s")
        wid = sid * _NC + cid
        HA = [ha0, ha1]
        HB = [hb0, hb1]
        PE = [pe0, pe1]
        INS = [ins0, ins1]
        GS = [gs0, gs1]
        OUTS = [outs0, outs1]
        ebase = wid * _EPW

        def issue_in(g0, c, s):
            eb = ebase + (g0 + c) * _B
            return [
                pltpu.async_copy(pe_h.at[pl.ds(eb, _B)], PE[s], INS[s]),
                pltpu.async_copy(hahb_h.at[src_v.at[pl.ds(c * _B, _B)]],
                                 HA[s], GS[s]),
                pltpu.async_copy(hahb_h.at[dst_v.at[pl.ds(c * _B, _B)]],
                                 HB[s], GS[s]),
            ]

        def issue_out(g0, c, s):
            eb = ebase + (g0 + c) * _B
            return pltpu.async_copy(PE[s], q_h.at[pl.ds(eb, _B)], OUTS[s])

        def compute(s):
            @pl.loop(0, _B, step=2)
            def _(r0):
                for dr in range(2):
                    r = r0 + dr
                    for j in range(4):
                        sl = pl.ds(16 * j, 16)
                        sh = pl.ds(D + 16 * j, 16)
                        PE[s][r, sl] = jnp.maximum(
                            PE[s][r, sl] + HA[s][r, sl] + HB[s][r, sh], 0.0)

        @pl.loop(0, _NPH)
        def _(ph):
            g0 = ph * _CPP
            eb0 = ebase + g0 * _B
            pltpu.sync_copy(src_h.at[pl.ds(eb0, _CPP * _B)], src_v)
            pltpu.sync_copy(dst_h.at[pl.ds(eb0, _CPP * _B)], dst_v)

            @pl.loop(0, _CPP, step=2)
            def _(t):
                din0 = issue_in(g0, t, 0)
                din1 = issue_in(g0, t + 1, 1)
                for d in din0:
                    d.wait()
                compute(0)
                dout0 = issue_out(g0, t, 0)
                for d in din1:
                    d.wait()
                compute(1)
                dout1 = issue_out(g0, t + 1, 1)
                dout0.wait()
                dout1.wait()

    return k(src, dst, pe, hahb)


def kernel(edge_index, x, e, params):
    src = edge_index[0]
    dst = edge_index[1]
    p = params
    gnn = p['gnn']

    def wb(layer):
        wcat = jnp.concatenate(
            [layer['B_W'], layer['D_W'], layer['E_W'], layer['A_W']], axis=1)
        bcat = jnp.concatenate(
            [layer['B_b'], layer['D_b'], layer['E_b'], layer['A_b']])
        return wcat, bcat

    wcat0, bcat0 = wb(gnn[0])
    h, bd, ea, ee, ce = _encoders(
        x, p['enc_W1'], p['enc_b1'], p['enc_W2'], p['enc_b2'], wcat0, bcat0,
        e, p['e1_W'], p['e1_b'], p['e2_W'], p['e2_b'],
        gnn[0]['C_W'], gnn[0]['C_b'])

    for li in range(len(gnn) - 1):
        ee, ndpart = _sc_layer(src, dst, ce, ee, bd, ea)
        wcat, bcat = wb(gnn[li + 1])
        h, bd, ea, ce = _interlayer(
            h, ea, ndpart, wcat, bcat,
            gnn[li + 1]['C_W'], gnn[li + 1]['C_b'], ee)

    ee, ndpart = _sc_layer(src, dst, ce, ee, bd, ea)
    w1 = p['sp_W1']
    hahb, pe = _final_prep(
        h, ea, ndpart, jnp.concatenate([w1[:D], w1[D:2 * D]], axis=1),
        w1[2 * D:], p['sp_b1'], ee)
    q = _sc_final(src, dst, pe, hahb)
    scores = _matmul_bias(q, p['sp_W2'], p['sp_b2'], 2000)
    return scores


# speedup vs baseline: 241.7191x; 1.0118x over previous
import functools

import jax
import jax.numpy as jnp
from jax import lax
from jax.experimental import pallas as pl
from jax.experimental.pallas import tpu as pltpu
from jax.experimental.pallas import tpu_sc as plsc

N = 10000
E = 320000
D = 64

_INTERPRET = False


def _mlp2_body(x_ref, w1_ref, b1_ref, w2_ref, b2_ref, o_ref):
    h = jnp.maximum(
        jnp.dot(x_ref[...], w1_ref[...], preferred_element_type=jnp.float32)
        + b1_ref[...], 0.0)
    o_ref[...] = (
        jnp.dot(h, w2_ref[...], preferred_element_type=jnp.float32)
        + b2_ref[...])


def _mlp2(x, w1, b1, w2, b2, blk):
    rows = x.shape[0]
    f_in, f_mid = w1.shape
    f_out = w2.shape[1]
    grid = rows // blk
    return pl.pallas_call(
        _mlp2_body,
        grid=(grid,),
        in_specs=[
            pl.BlockSpec((blk, f_in), lambda i: (i, 0)),
            pl.BlockSpec((f_in, f_mid), lambda i: (0, 0)),
            pl.BlockSpec((1, f_mid), lambda i: (0, 0)),
            pl.BlockSpec((f_mid, f_out), lambda i: (0, 0)),
            pl.BlockSpec((1, f_out), lambda i: (0, 0)),
        ],
        out_specs=pl.BlockSpec((blk, f_out), lambda i: (i, 0)),
        out_shape=jax.ShapeDtypeStruct((rows, f_out), jnp.float32),
        interpret=_INTERPRET,
    )(x, w1, b1.reshape(1, -1), w2, b2.reshape(1, -1))


def _matmul_bias_body(x_ref, w_ref, b_ref, o_ref):
    o_ref[...] = (
        jnp.dot(x_ref[...], w_ref[...], preferred_element_type=jnp.float32)
        + b_ref[...])


def _matmul_bias(x, w, b, blk):
    rows = x.shape[0]
    f_in, f_out = w.shape
    grid = rows // blk
    return pl.pallas_call(
        _matmul_bias_body,
        grid=(grid,),
        in_specs=[
            pl.BlockSpec((blk, f_in), lambda i: (i, 0)),
            pl.BlockSpec((f_in, f_out), lambda i: (0, 0)),
            pl.BlockSpec((1, f_out), lambda i: (0, 0)),
        ],
        out_specs=pl.BlockSpec((blk, f_out), lambda i: (i, 0)),
        out_shape=jax.ShapeDtypeStruct((rows, f_out), jnp.float32),
        interpret=_INTERPRET,
    )(x, w, b.reshape(1, -1))


def _node_mm2_body(x_ref, w_ref, b_ref, o1_ref, o2_ref):
    o = (jnp.dot(x_ref[...], w_ref[...], preferred_element_type=jnp.float32)
         + b_ref[...])
    o1_ref[...] = o[:, :2 * D]
    o2_ref[...] = o[:, 2 * D:]


def _node_mm2(x, w, b, blk):
    grid = N // blk
    return pl.pallas_call(
        _node_mm2_body,
        grid=(grid,),
        in_specs=[
            pl.BlockSpec((blk, D), lambda i: (i, 0)),
            pl.BlockSpec((D, 4 * D), lambda i: (0, 0)),
            pl.BlockSpec((1, 4 * D), lambda i: (0, 0)),
        ],
        out_specs=[
            pl.BlockSpec((blk, 2 * D), lambda i: (i, 0)),
            pl.BlockSpec((blk, 2 * D), lambda i: (i, 0)),
        ],
        out_shape=[
            jax.ShapeDtypeStruct((N, 2 * D), jnp.float32),
            jax.ShapeDtypeStruct((N, 2 * D), jnp.float32),
        ],
        interpret=_INTERPRET,
    )(x, w, b.reshape(1, -1))


_NB = 10
_EBLK = 2000
_NBLK = 1000


def _nodeidx(i):
    return (jnp.minimum(i, _NB - 1), 0)


def _upd(h_ref, eap_ref, nd_ref):
    num = nd_ref[0, :, :D] + nd_ref[1, :, :D]
    den = nd_ref[0, :, D:] + nd_ref[1, :, D:] + 1e-6
    return h_ref[...] + jnp.maximum(eap_ref[:, D:] + num / den, 0.0)


def _interlayer_body(h_ref, eap_ref, nd_ref, wcat_ref, bcat_ref,
                     cw_ref, cb_ref, ee_ref,
                     hn_ref, bd_ref, ean_ref, ce_ref):
    i = pl.program_id(0)

    @pl.when(i < _NB)
    def _():
        hn = _upd(h_ref, eap_ref, nd_ref)
        hn_ref[...] = hn
        o = (jnp.dot(hn, wcat_ref[...], preferred_element_type=jnp.float32)
             + bcat_ref[...])
        bd_ref[...] = o[:, :2 * D]
        ean_ref[...] = o[:, 2 * D:]

    ce_ref[...] = (
        jnp.dot(ee_ref[...], cw_ref[...], preferred_element_type=jnp.float32)
        + cb_ref[...])


def _interlayer(h, ea_prev, nd, wcat, bcat, cw, cb, ee):
    return pl.pallas_call(
        _interlayer_body,
        grid=(E // _EBLK,),
        in_specs=[
            pl.BlockSpec((_NBLK, D), _nodeidx),
            pl.BlockSpec((_NBLK, 2 * D), _nodeidx),
            pl.BlockSpec((2, _NBLK, 2 * D), lambda i: (0, *_nodeidx(i))),
            pl.BlockSpec((D, 4 * D), lambda i: (0, 0)),
            pl.BlockSpec((1, 4 * D), lambda i: (0, 0)),
            pl.BlockSpec((D, D), lambda i: (0, 0)),
            pl.BlockSpec((1, D), lambda i: (0, 0)),
            pl.BlockSpec((_EBLK, D), lambda i: (i, 0)),
        ],
        out_specs=[
            pl.BlockSpec((_NBLK, D), _nodeidx),
            pl.BlockSpec((_NBLK, 2 * D), _nodeidx),
            pl.BlockSpec((_NBLK, 2 * D), _nodeidx),
            pl.BlockSpec((_EBLK, D), lambda i: (i, 0)),
        ],
        out_shape=[
            jax.ShapeDtypeStruct((N, D), jnp.float32),
            jax.ShapeDtypeStruct((N, 2 * D), jnp.float32),
            jax.ShapeDtypeStruct((N, 2 * D), jnp.float32),
            jax.ShapeDtypeStruct((E, D), jnp.float32),
        ],
        interpret=_INTERPRET,
    )(h, ea_prev, nd, wcat, bcat.reshape(1, -1), cw, cb.reshape(1, -1), ee)


def _final_prep_body(h_ref, eap_ref, nd_ref, w12_ref, w1c_ref, b1_ref,
                     ee_ref, hahb_ref, pe_ref):
    i = pl.program_id(0)

    @pl.when(i < _NB)
    def _():
        hn = _upd(h_ref, eap_ref, nd_ref)
        hahb_ref[...] = jnp.dot(hn, w12_ref[...],
                                preferred_element_type=jnp.float32)

    pe_ref[...] = (
        jnp.dot(ee_ref[...], w1c_ref[...], preferred_element_type=jnp.float32)
        + b1_ref[...])


def _final_prep(h, ea_prev, nd, w12, w1c, b1, ee):
    return pl.pallas_call(
        _final_prep_body,
        grid=(E // _EBLK,),
        in_specs=[
            pl.BlockSpec((_NBLK, D), _nodeidx),
            pl.BlockSpec((_NBLK, 2 * D), _nodeidx),
            pl.BlockSpec((2, _NBLK, 2 * D), lambda i: (0, *_nodeidx(i))),
            pl.BlockSpec((D, 2 * D), lambda i: (0, 0)),
            pl.BlockSpec((D, D), lambda i: (0, 0)),
            pl.BlockSpec((1, D), lambda i: (0, 0)),
            pl.BlockSpec((_EBLK, D), lambda i: (i, 0)),
        ],
        out_specs=[
            pl.BlockSpec((_NBLK, 2 * D), _nodeidx),
            pl.BlockSpec((_EBLK, D), lambda i: (i, 0)),
        ],
        out_shape=[
            jax.ShapeDtypeStruct((N, 2 * D), jnp.float32),
            jax.ShapeDtypeStruct((E, D), jnp.float32),
        ],
        interpret=_INTERPRET,
    )(h, ea_prev, nd, w12, w1c, b1.reshape(1, -1), ee)


def _encoders_body(x_ref, nw1_ref, nb1_ref, nw2_ref, nb2_ref,
                   wcat_ref, bcat_ref,
                   e_ref, ew1_ref, eb1_ref, ew2_ref, eb2_ref,
                   cw_ref, cb_ref,
                   h_ref, bd_ref, ea_ref, ee_ref, ce_ref):
    i = pl.program_id(0)

    @pl.when(i < _NB)
    def _():
        t = jnp.maximum(
            jnp.dot(x_ref[...], nw1_ref[...],
                    preferred_element_type=jnp.float32) + nb1_ref[...], 0.0)
        h = (jnp.dot(t, nw2_ref[...], preferred_element_type=jnp.float32)
             + nb2_ref[...])
        h_ref[...] = h
        o = (jnp.dot(h, wcat_ref[...], preferred_element_type=jnp.float32)
             + bcat_ref[...])
        bd_ref[...] = o[:, :2 * D]
        ea_ref[...] = o[:, 2 * D:]

    t = jnp.maximum(
        jnp.dot(e_ref[...], ew1_ref[...],
                preferred_element_type=jnp.float32) + eb1_ref[...], 0.0)
    ee = (jnp.dot(t, ew2_ref[...], preferred_element_type=jnp.float32)
          + eb2_ref[...])
    ee_ref[...] = ee
    ce_ref[...] = (
        jnp.dot(ee, cw_ref[...], preferred_element_type=jnp.float32)
        + cb_ref[...])


def _encoders(x, nw1, nb1, nw2, nb2, wcat, bcat,
              e, ew1, eb1, ew2, eb2, cw, cb):
    hid = ew1.shape[1]
    return pl.pallas_call(
        _encoders_body,
        grid=(E // _EBLK,),
        in_specs=[
            pl.BlockSpec((_NBLK, x.shape[1]), _nodeidx),
            pl.BlockSpec((x.shape[1], D), lambda i: (0, 0)),
            pl.BlockSpec((1, D), lambda i: (0, 0)),
            pl.BlockSpec((D, D), lambda i: (0, 0)),
            pl.BlockSpec((1, D), lambda i: (0, 0)),
            pl.BlockSpec((D, 4 * D), lambda i: (0, 0)),
            pl.BlockSpec((1, 4 * D), lambda i: (0, 0)),
            pl.BlockSpec((_EBLK, e.shape[1]), lambda i: (i, 0)),
            pl.BlockSpec((e.shape[1], hid), lambda i: (0, 0)),
            pl.BlockSpec((1, hid), lambda i: (0, 0)),
            pl.BlockSpec((hid, D), lambda i: (0, 0)),
            pl.BlockSpec((1, D), lambda i: (0, 0)),
            pl.BlockSpec((D, D), lambda i: (0, 0)),
            pl.BlockSpec((1, D), lambda i: (0, 0)),
        ],
        out_specs=[
            pl.BlockSpec((_NBLK, D), _nodeidx),
            pl.BlockSpec((_NBLK, 2 * D), _nodeidx),
            pl.BlockSpec((_NBLK, 2 * D), _nodeidx),
            pl.BlockSpec((_EBLK, D), lambda i: (i, 0)),
            pl.BlockSpec((_EBLK, D), lambda i: (i, 0)),
        ],
        out_shape=[
            jax.ShapeDtypeStruct((N, D), jnp.float32),
            jax.ShapeDtypeStruct((N, 2 * D), jnp.float32),
            jax.ShapeDtypeStruct((N, 2 * D), jnp.float32),
            jax.ShapeDtypeStruct((E, D), jnp.float32),
            jax.ShapeDtypeStruct((E, D), jnp.float32),
        ],
        interpret=_INTERPRET,
    )(x, nw1, nb1.reshape(1, -1), nw2, nb2.reshape(1, -1),
      wcat, bcat.reshape(1, -1),
      e, ew1, eb1.reshape(1, -1), ew2, eb2.reshape(1, -1),
      cw, cb.reshape(1, -1))


_NC = 2
_NS = 16
_NW = _NC * _NS
_B = 40
_EPW = E // _NW
_TPW = _EPW // _B
_NPH = 5
_CPP = _TPW // _NPH
_NCH = N // _B


def _sc_mesh():
    return plsc.VectorSubcoreMesh(
        core_axis_name="c", subcore_axis_name="s",
        num_cores=_NC, num_subcores=_NS)


def _sc_layer(src, dst, ce, ee, bd, ea):
    @functools.partial(
        pl.kernel,
        out_type=[jax.ShapeDtypeStruct((E, D), jnp.float32),
                  jax.ShapeDtypeStruct((_NC, N, 2 * D), jnp.float32)],
        mesh=_sc_mesh(),
        scratch_types=[
            pltpu.VMEM((_CPP * _B,), jnp.int32),
            pltpu.VMEM((_CPP * _B,), jnp.int32),
            pltpu.VMEM((_B,), jnp.int32),
            pltpu.VMEM((_B,), jnp.int32),
            pltpu.VMEM((_B, 2 * D), jnp.float32),
            pltpu.VMEM((_B, 2 * D), jnp.float32),
            pltpu.VMEM((_B, 2 * D), jnp.float32),
            pltpu.VMEM((_B, 2 * D), jnp.float32),
            pltpu.VMEM((_B, D), jnp.float32),
            pltpu.VMEM((_B, D), jnp.float32),
            pltpu.VMEM((_B, D), jnp.float32),
            pltpu.VMEM((_B, D), jnp.float32),
            pltpu.VMEM_SHARED((N, 2 * D), jnp.float32),
            pltpu.SemaphoreType.DMA,
            pltpu.SemaphoreType.DMA,
            pltpu.SemaphoreType.DMA,
            pltpu.SemaphoreType.DMA,
            pltpu.SemaphoreType.DMA,
            pltpu.SemaphoreType.DMA,
        ])
    def k(src_h, dst_h, ce_h, ee_h, bd_h, ea_h, eeo_h, nd_h,
          src_v, dst_v, idst0, idst1, bd0, bd1, ea0, ea1,
          ce0, ce1, ee0, ee1, nd_sh, ins0, ins1, gs0, gs1, outs0, outs1):
        cid = lax.axis_index("c")
        sid = lax.axis_index("s")
        wid = sid * _NC + cid
        IDST = [idst0, idst1]
        BD = [bd0, bd1]
        EA = [ea0, ea1]
        CE = [ce0, ce1]
        EE = [ee0, ee1]
        INS = [ins0, ins1]
        GS = [gs0, gs1]
        OUTS = [outs0, outs1]
        ebase = wid * _EPW

        @pl.loop(0, _B)
        def _(r):
            for j in range(8):
                bd0[r, pl.ds(16 * j, 16)] = jnp.zeros((16,), jnp.float32)

        @pl.loop(0, pl.cdiv(_NCH, _NS))
        def _(t):
            c = sid + t * _NS

            @pl.when(c < _NCH)
            def _():
                pltpu.sync_copy(bd0, nd_sh.at[pl.ds(c * _B, _B)])

        plsc.subcore_barrier()

        def issue_in(g0, c, s):
            eb = ebase + (g0 + c) * _B
            return [
                pltpu.async_copy(ce_h.at[pl.ds(eb, _B)], CE[s], INS[s]),
                pltpu.async_copy(ee_h.at[pl.ds(eb, _B)], EE[s], INS[s]),
                pltpu.async_copy(dst_h.at[pl.ds(eb, _B)], IDST[s], INS[s]),
                pltpu.async_copy(bd_h.at[src_v.at[pl.ds(c * _B, _B)]],
                                 BD[s], GS[s]),
                pltpu.async_copy(ea_h.at[dst_v.at[pl.ds(c * _B, _B)]],
                                 EA[s], GS[s]),
            ]

        def issue_out(g0, c, s):
            eb = ebase + (g0 + c) * _B
            return [
                pltpu.async_copy(EE[s], eeo_h.at[pl.ds(eb, _B)], OUTS[s]),
                pltpu.async_copy(BD[s], nd_sh.at[IDST[s]], GS[s], add=True),
            ]

        def compute(s):
            @pl.loop(0, _B, step=4)
            def _(r0):
                for dr in range(4):
                    r = r0 + dr
                    for j in range(4):
                        sl = pl.ds(16 * j, 16)
                        sh = pl.ds(D + 16 * j, 16)
                        bv = BD[s][r, sl]
                        ehat = CE[s][r, sl] + BD[s][r, sh] + EA[s][r, sl]
                        sg = 1.0 / (1.0 + jnp.exp(-ehat))
                        BD[s][r, sl] = sg * bv
                        BD[s][r, sh] = sg
                        EE[s][r, sl] = EE[s][r, sl] + jnp.maximum(ehat, 0.0)

        @pl.loop(0, _NPH)
        def _(ph):
            g0 = ph * _CPP
            eb0 = ebase + g0 * _B
            d1 = pltpu.async_copy(src_h.at[pl.ds(eb0, _CPP * _B)], src_v,
                                  GS[0])
            d2 = pltpu.async_copy(dst_h.at[pl.ds(eb0, _CPP * _B)], dst_v,
                                  GS[1])
            d1.wait()
            d2.wait()

            @pl.loop(0, _CPP, step=2)
            def _(t):
                din0 = issue_in(g0, t, 0)
                din1 = issue_in(g0, t + 1, 1)
                for d in din0:
                    d.wait()
                compute(0)
                dout0 = issue_out(g0, t, 0)
                for d in din1:
                    d.wait()
                compute(1)
                dout1 = issue_out(g0, t + 1, 1)
                for d in dout0:
                    d.wait()
                for d in dout1:
                    d.wait()

        plsc.subcore_barrier()

        @pl.when(sid == 0)
        def _():
            pltpu.async_copy(nd_sh, nd_h.at[cid], OUTS[0]).wait()

    return k(src, dst, ce, ee, bd, ea)


def _sc_final(src, dst, pe, hahb):
    @functools.partial(
        pl.kernel,
        out_type=jax.ShapeDtypeStruct((E, D), jnp.float32),
        mesh=_sc_mesh(),
        scratch_types=[
            pltpu.VMEM((_CPP * _B,), jnp.int32),
            pltpu.VMEM((_CPP * _B,), jnp.int32),
            pltpu.VMEM((_B, 2 * D), jnp.float32),
            pltpu.VMEM((_B, 2 * D), jnp.float32),
            pltpu.VMEM((_B, 2 * D), jnp.float32),
            pltpu.VMEM((_B, 2 * D), jnp.float32),
            pltpu.VMEM((_B, D), jnp.float32),
            pltpu.VMEM((_B, D), jnp.float32),
            pltpu.SemaphoreType.DMA,
            pltpu.SemaphoreType.DMA,
            pltpu.SemaphoreType.DMA,
            pltpu.SemaphoreType.DMA,
            pltpu.SemaphoreType.DMA,
            pltpu.SemaphoreType.DMA,
        ])
    def k(src_h, dst_h, pe_h, hahb_h, q_h,
          src_v, dst_v, ha0, ha1, hb0, hb1, pe0, pe1,
          ins0, ins1, gs0, gs1, outs0, outs1):
        cid = lax.axis_index("c")
        sid = lax.axis_index("s")
        wid = sid * _NC + cid
        HA = [ha0, ha1]
        HB = [hb0, hb1]
        PE = [pe0, pe1]
        INS = [ins0, ins1]
        GS = [gs0, gs1]
        OUTS = [outs0, outs1]
        ebase = wid * _EPW

        def issue_in(g0, c, s):
            eb = ebase + (g0 + c) * _B
            return [
                pltpu.async_copy(pe_h.at[pl.ds(eb, _B)], PE[s], INS[s]),
                pltpu.async_copy(hahb_h.at[src_v.at[pl.ds(c * _B, _B)]],
                                 HA[s], GS[s]),
                pltpu.async_copy(hahb_h.at[dst_v.at[pl.ds(c * _B, _B)]],
                                 HB[s], GS[s]),
            ]

        def issue_out(g0, c, s):
            eb = ebase + (g0 + c) * _B
            return pltpu.async_copy(PE[s], q_h.at[pl.ds(eb, _B)], OUTS[s])

        def compute(s):
            @pl.loop(0, _B, step=4)
            def _(r0):
                for dr in range(4):
                    r = r0 + dr
                    for j in range(4):
                        sl = pl.ds(16 * j, 16)
                        sh = pl.ds(D + 16 * j, 16)
                        PE[s][r, sl] = jnp.maximum(
                            PE[s][r, sl] + HA[s][r, sl] + HB[s][r, sh], 0.0)

        @pl.loop(0, _NPH)
        def _(ph):
            g0 = ph * _CPP
            eb0 = ebase + g0 * _B
            d1 = pltpu.async_copy(src_h.at[pl.ds(eb0, _CPP * _B)], src_v,
                                  GS[0])
            d2 = pltpu.async_copy(dst_h.at[pl.ds(eb0, _CPP * _B)], dst_v,
                                  GS[1])
            d1.wait()
            d2.wait()

            @pl.loop(0, _CPP, step=2)
            def _(t):
                din0 = issue_in(g0, t, 0)
                din1 = issue_in(g0, t + 1, 1)
                for d in din0:
                    d.wait()
                compute(0)
                dout0 = issue_out(g0, t, 0)
                for d in din1:
                    d.wait()
                compute(1)
                dout1 = issue_out(g0, t + 1, 1)
                dout0.wait()
                dout1.wait()

    return k(src, dst, pe, hahb)


def kernel(edge_index, x, e, params):
    src = edge_index[0]
    dst = edge_index[1]
    p = params
    gnn = p['gnn']

    def wb(layer):
        wcat = jnp.concatenate(
            [layer['B_W'], layer['D_W'], layer['E_W'], layer['A_W']], axis=1)
        bcat = jnp.concatenate(
            [layer['B_b'], layer['D_b'], layer['E_b'], layer['A_b']])
        return wcat, bcat

    wcat0, bcat0 = wb(gnn[0])
    h, bd, ea, ee, ce = _encoders(
        x, p['enc_W1'], p['enc_b1'], p['enc_W2'], p['enc_b2'], wcat0, bcat0,
        e, p['e1_W'], p['e1_b'], p['e2_W'], p['e2_b'],
        gnn[0]['C_W'], gnn[0]['C_b'])

    for li in range(len(gnn) - 1):
        ee, ndpart = _sc_layer(src, dst, ce, ee, bd, ea)
        wcat, bcat = wb(gnn[li + 1])
        h, bd, ea, ce = _interlayer(
            h, ea, ndpart, wcat, bcat,
            gnn[li + 1]['C_W'], gnn[li + 1]['C_b'], ee)

    ee, ndpart = _sc_layer(src, dst, ce, ee, bd, ea)
    w1 = p['sp_W1']
    hahb, pe = _final_prep(
        h, ea, ndpart, jnp.concatenate([w1[:D], w1[D:2 * D]], axis=1),
        w1[2 * D:], p['sp_b1'], ee)
    q = _sc_final(src, dst, pe, hahb)
    scores = _matmul_bias(q, p['sp_W2'], p['sp_b2'], 2000)
    return scores


# speedup vs baseline: 247.8941x; 1.0255x over previous
import functools

import jax
import jax.numpy as jnp
from jax import lax
from jax.experimental import pallas as pl
from jax.experimental.pallas import tpu as pltpu
from jax.experimental.pallas import tpu_sc as plsc

N = 10000
E = 320000
D = 64

_INTERPRET = False


def _mlp2_body(x_ref, w1_ref, b1_ref, w2_ref, b2_ref, o_ref):
    h = jnp.maximum(
        jnp.dot(x_ref[...], w1_ref[...], preferred_element_type=jnp.float32)
        + b1_ref[...], 0.0)
    o_ref[...] = (
        jnp.dot(h, w2_ref[...], preferred_element_type=jnp.float32)
        + b2_ref[...])


def _mlp2(x, w1, b1, w2, b2, blk):
    rows = x.shape[0]
    f_in, f_mid = w1.shape
    f_out = w2.shape[1]
    grid = rows // blk
    return pl.pallas_call(
        _mlp2_body,
        grid=(grid,),
        in_specs=[
            pl.BlockSpec((blk, f_in), lambda i: (i, 0)),
            pl.BlockSpec((f_in, f_mid), lambda i: (0, 0)),
            pl.BlockSpec((1, f_mid), lambda i: (0, 0)),
            pl.BlockSpec((f_mid, f_out), lambda i: (0, 0)),
            pl.BlockSpec((1, f_out), lambda i: (0, 0)),
        ],
        out_specs=pl.BlockSpec((blk, f_out), lambda i: (i, 0)),
        out_shape=jax.ShapeDtypeStruct((rows, f_out), jnp.float32),
        interpret=_INTERPRET,
    )(x, w1, b1.reshape(1, -1), w2, b2.reshape(1, -1))


def _matmul_bias_body(x_ref, w_ref, b_ref, o_ref):
    o_ref[...] = (
        jnp.dot(x_ref[...], w_ref[...], preferred_element_type=jnp.float32)
        + b_ref[...])


def _matmul_bias(x, w, b, blk):
    rows = x.shape[0]
    f_in, f_out = w.shape
    grid = rows // blk
    return pl.pallas_call(
        _matmul_bias_body,
        grid=(grid,),
        in_specs=[
            pl.BlockSpec((blk, f_in), lambda i: (i, 0)),
            pl.BlockSpec((f_in, f_out), lambda i: (0, 0)),
            pl.BlockSpec((1, f_out), lambda i: (0, 0)),
        ],
        out_specs=pl.BlockSpec((blk, f_out), lambda i: (i, 0)),
        out_shape=jax.ShapeDtypeStruct((rows, f_out), jnp.float32),
        interpret=_INTERPRET,
    )(x, w, b.reshape(1, -1))


def _node_mm2_body(x_ref, w_ref, b_ref, o1_ref, o2_ref):
    o = (jnp.dot(x_ref[...], w_ref[...], preferred_element_type=jnp.float32)
         + b_ref[...])
    o1_ref[...] = o[:, :2 * D]
    o2_ref[...] = o[:, 2 * D:]


def _node_mm2(x, w, b, blk):
    grid = N // blk
    return pl.pallas_call(
        _node_mm2_body,
        grid=(grid,),
        in_specs=[
            pl.BlockSpec((blk, D), lambda i: (i, 0)),
            pl.BlockSpec((D, 4 * D), lambda i: (0, 0)),
            pl.BlockSpec((1, 4 * D), lambda i: (0, 0)),
        ],
        out_specs=[
            pl.BlockSpec((blk, 2 * D), lambda i: (i, 0)),
            pl.BlockSpec((blk, 2 * D), lambda i: (i, 0)),
        ],
        out_shape=[
            jax.ShapeDtypeStruct((N, 2 * D), jnp.float32),
            jax.ShapeDtypeStruct((N, 2 * D), jnp.float32),
        ],
        interpret=_INTERPRET,
    )(x, w, b.reshape(1, -1))


_NB = 10
_EBLK = 2000
_NBLK = 1000


def _nodeidx(i):
    return (jnp.minimum(i, _NB - 1), 0)


def _upd(h_ref, eap_ref, nd_ref):
    num = nd_ref[0, :, :D] + nd_ref[1, :, :D]
    den = nd_ref[0, :, D:] + nd_ref[1, :, D:] + 1e-6
    return h_ref[...] + jnp.maximum(eap_ref[:, D:] + num / den, 0.0)


def _interlayer_body(h_ref, eap_ref, nd_ref, wcat_ref, bcat_ref,
                     cw_ref, cb_ref, ee_ref, r_ref,
                     hn_ref, bd_ref, ean_ref, een_ref, ce_ref):
    i = pl.program_id(0)

    @pl.when(i < _NB)
    def _():
        hn = _upd(h_ref, eap_ref, nd_ref)
        hn_ref[...] = hn
        o = (jnp.dot(hn, wcat_ref[...], preferred_element_type=jnp.float32)
             + bcat_ref[...])
        bd_ref[...] = o[:, :2 * D]
        ean_ref[...] = o[:, 2 * D:]

    een = ee_ref[...] + r_ref[...]
    een_ref[...] = een
    ce_ref[...] = (
        jnp.dot(een, cw_ref[...], preferred_element_type=jnp.float32)
        + cb_ref[...])


def _interlayer(h, ea_prev, nd, wcat, bcat, cw, cb, ee, r):
    return pl.pallas_call(
        _interlayer_body,
        grid=(E // _EBLK,),
        in_specs=[
            pl.BlockSpec((_NBLK, D), _nodeidx),
            pl.BlockSpec((_NBLK, 2 * D), _nodeidx),
            pl.BlockSpec((2, _NBLK, 2 * D), lambda i: (0, *_nodeidx(i))),
            pl.BlockSpec((D, 4 * D), lambda i: (0, 0)),
            pl.BlockSpec((1, 4 * D), lambda i: (0, 0)),
            pl.BlockSpec((D, D), lambda i: (0, 0)),
            pl.BlockSpec((1, D), lambda i: (0, 0)),
            pl.BlockSpec((_EBLK, D), lambda i: (i, 0)),
            pl.BlockSpec((_EBLK, D), lambda i: (i, 0)),
        ],
        out_specs=[
            pl.BlockSpec((_NBLK, D), _nodeidx),
            pl.BlockSpec((_NBLK, 2 * D), _nodeidx),
            pl.BlockSpec((_NBLK, 2 * D), _nodeidx),
            pl.BlockSpec((_EBLK, D), lambda i: (i, 0)),
            pl.BlockSpec((_EBLK, D), lambda i: (i, 0)),
        ],
        out_shape=[
            jax.ShapeDtypeStruct((N, D), jnp.float32),
            jax.ShapeDtypeStruct((N, 2 * D), jnp.float32),
            jax.ShapeDtypeStruct((N, 2 * D), jnp.float32),
            jax.ShapeDtypeStruct((E, D), jnp.float32),
            jax.ShapeDtypeStruct((E, D), jnp.float32),
        ],
        interpret=_INTERPRET,
    )(h, ea_prev, nd, wcat, bcat.reshape(1, -1), cw, cb.reshape(1, -1),
      ee, r)


def _final_prep_body(h_ref, eap_ref, nd_ref, w12_ref, w1c_ref, b1_ref,
                     ee_ref, r_ref, hahb_ref, pe_ref):
    i = pl.program_id(0)

    @pl.when(i < _NB)
    def _():
        hn = _upd(h_ref, eap_ref, nd_ref)
        hahb_ref[...] = jnp.dot(hn, w12_ref[...],
                                preferred_element_type=jnp.float32)

    pe_ref[...] = (
        jnp.dot(ee_ref[...] + r_ref[...], w1c_ref[...],
                preferred_element_type=jnp.float32)
        + b1_ref[...])


def _final_prep(h, ea_prev, nd, w12, w1c, b1, ee, r):
    return pl.pallas_call(
        _final_prep_body,
        grid=(E // _EBLK,),
        in_specs=[
            pl.BlockSpec((_NBLK, D), _nodeidx),
            pl.BlockSpec((_NBLK, 2 * D), _nodeidx),
            pl.BlockSpec((2, _NBLK, 2 * D), lambda i: (0, *_nodeidx(i))),
            pl.BlockSpec((D, 2 * D), lambda i: (0, 0)),
            pl.BlockSpec((D, D), lambda i: (0, 0)),
            pl.BlockSpec((1, D), lambda i: (0, 0)),
            pl.BlockSpec((_EBLK, D), lambda i: (i, 0)),
            pl.BlockSpec((_EBLK, D), lambda i: (i, 0)),
        ],
        out_specs=[
            pl.BlockSpec((_NBLK, 2 * D), _nodeidx),
            pl.BlockSpec((_EBLK, D), lambda i: (i, 0)),
        ],
        out_shape=[
            jax.ShapeDtypeStruct((N, 2 * D), jnp.float32),
            jax.ShapeDtypeStruct((E, D), jnp.float32),
        ],
        interpret=_INTERPRET,
    )(h, ea_prev, nd, w12, w1c, b1.reshape(1, -1), ee, r)


def _encoders_body(x_ref, nw1_ref, nb1_ref, nw2_ref, nb2_ref,
                   wcat_ref, bcat_ref,
                   e_ref, ew1_ref, eb1_ref, ew2_ref, eb2_ref,
                   cw_ref, cb_ref,
                   h_ref, bd_ref, ea_ref, ee_ref, ce_ref):
    i = pl.program_id(0)

    @pl.when(i < _NB)
    def _():
        t = jnp.maximum(
            jnp.dot(x_ref[...], nw1_ref[...],
                    preferred_element_type=jnp.float32) + nb1_ref[...], 0.0)
        h = (jnp.dot(t, nw2_ref[...], preferred_element_type=jnp.float32)
             + nb2_ref[...])
        h_ref[...] = h
        o = (jnp.dot(h, wcat_ref[...], preferred_element_type=jnp.float32)
             + bcat_ref[...])
        bd_ref[...] = o[:, :2 * D]
        ea_ref[...] = o[:, 2 * D:]

    t = jnp.maximum(
        jnp.dot(e_ref[...], ew1_ref[...],
                preferred_element_type=jnp.float32) + eb1_ref[...], 0.0)
    ee = (jnp.dot(t, ew2_ref[...], preferred_element_type=jnp.float32)
          + eb2_ref[...])
    ee_ref[...] = ee
    ce_ref[...] = (
        jnp.dot(ee, cw_ref[...], preferred_element_type=jnp.float32)
        + cb_ref[...])


def _encoders(x, nw1, nb1, nw2, nb2, wcat, bcat,
              e, ew1, eb1, ew2, eb2, cw, cb):
    hid = ew1.shape[1]
    return pl.pallas_call(
        _encoders_body,
        grid=(E // _EBLK,),
        in_specs=[
            pl.BlockSpec((_NBLK, x.shape[1]), _nodeidx),
            pl.BlockSpec((x.shape[1], D), lambda i: (0, 0)),
            pl.BlockSpec((1, D), lambda i: (0, 0)),
            pl.BlockSpec((D, D), lambda i: (0, 0)),
            pl.BlockSpec((1, D), lambda i: (0, 0)),
            pl.BlockSpec((D, 4 * D), lambda i: (0, 0)),
            pl.BlockSpec((1, 4 * D), lambda i: (0, 0)),
            pl.BlockSpec((_EBLK, e.shape[1]), lambda i: (i, 0)),
            pl.BlockSpec((e.shape[1], hid), lambda i: (0, 0)),
            pl.BlockSpec((1, hid), lambda i: (0, 0)),
            pl.BlockSpec((hid, D), lambda i: (0, 0)),
            pl.BlockSpec((1, D), lambda i: (0, 0)),
            pl.BlockSpec((D, D), lambda i: (0, 0)),
            pl.BlockSpec((1, D), lambda i: (0, 0)),
        ],
        out_specs=[
            pl.BlockSpec((_NBLK, D), _nodeidx),
            pl.BlockSpec((_NBLK, 2 * D), _nodeidx),
            pl.BlockSpec((_NBLK, 2 * D), _nodeidx),
            pl.BlockSpec((_EBLK, D), lambda i: (i, 0)),
            pl.BlockSpec((_EBLK, D), lambda i: (i, 0)),
        ],
        out_shape=[
            jax.ShapeDtypeStruct((N, D), jnp.float32),
            jax.ShapeDtypeStruct((N, 2 * D), jnp.float32),
            jax.ShapeDtypeStruct((N, 2 * D), jnp.float32),
            jax.ShapeDtypeStruct((E, D), jnp.float32),
            jax.ShapeDtypeStruct((E, D), jnp.float32),
        ],
        interpret=_INTERPRET,
    )(x, nw1, nb1.reshape(1, -1), nw2, nb2.reshape(1, -1),
      wcat, bcat.reshape(1, -1),
      e, ew1, eb1.reshape(1, -1), ew2, eb2.reshape(1, -1),
      cw, cb.reshape(1, -1))


_NC = 2
_NS = 16
_NW = _NC * _NS
_B = 40
_EPW = E // _NW
_TPW = _EPW // _B
_NPH = 5
_CPP = _TPW // _NPH
_NCH = N // _B


def _sc_mesh():
    return plsc.VectorSubcoreMesh(
        core_axis_name="c", subcore_axis_name="s",
        num_cores=_NC, num_subcores=_NS)


def _sc_layer(src, dst, ce, bd, ea):
    @functools.partial(
        pl.kernel,
        out_type=[jax.ShapeDtypeStruct((E, D), jnp.float32),
                  jax.ShapeDtypeStruct((_NC, N, 2 * D), jnp.float32)],
        mesh=_sc_mesh(),
        scratch_types=[
            pltpu.VMEM((_CPP * _B,), jnp.int32),
            pltpu.VMEM((_CPP * _B,), jnp.int32),
            pltpu.VMEM((_B,), jnp.int32),
            pltpu.VMEM((_B,), jnp.int32),
            pltpu.VMEM((_B, 2 * D), jnp.float32),
            pltpu.VMEM((_B, 2 * D), jnp.float32),
            pltpu.VMEM((_B, 2 * D), jnp.float32),
            pltpu.VMEM((_B, 2 * D), jnp.float32),
            pltpu.VMEM((_B, D), jnp.float32),
            pltpu.VMEM((_B, D), jnp.float32),
            pltpu.VMEM((_B, D), jnp.float32),
            pltpu.VMEM((_B, D), jnp.float32),
            pltpu.VMEM_SHARED((N, 2 * D), jnp.float32),
            pltpu.SemaphoreType.DMA,
            pltpu.SemaphoreType.DMA,
            pltpu.SemaphoreType.DMA,
            pltpu.SemaphoreType.DMA,
            pltpu.SemaphoreType.DMA,
            pltpu.SemaphoreType.DMA,
        ])
    def k(src_h, dst_h, ce_h, bd_h, ea_h, r_h, nd_h,
          src_v, dst_v, idst0, idst1, bd0, bd1, ea0, ea1,
          ce0, ce1, ee0, ee1, nd_sh, ins0, ins1, gs0, gs1, outs0, outs1):
        cid = lax.axis_index("c")
        sid = lax.axis_index("s")
        wid = sid * _NC + cid
        IDST = [idst0, idst1]
        BD = [bd0, bd1]
        EA = [ea0, ea1]
        CE = [ce0, ce1]
        EE = [ee0, ee1]
        INS = [ins0, ins1]
        GS = [gs0, gs1]
        OUTS = [outs0, outs1]
        ebase = wid * _EPW

        @pl.loop(0, _B)
        def _(r):
            for j in range(8):
                bd0[r, pl.ds(16 * j, 16)] = jnp.zeros((16,), jnp.float32)

        @pl.loop(0, pl.cdiv(_NCH, _NS))
        def _(t):
            c = sid + t * _NS

            @pl.when(c < _NCH)
            def _():
                pltpu.sync_copy(bd0, nd_sh.at[pl.ds(c * _B, _B)])

        plsc.subcore_barrier()

        def issue_in(g0, c, s):
            eb = ebase + (g0 + c) * _B
            return [
                pltpu.async_copy(ce_h.at[pl.ds(eb, _B)], CE[s], INS[s]),
                pltpu.async_copy(dst_h.at[pl.ds(eb, _B)], IDST[s], INS[s]),
                pltpu.async_copy(bd_h.at[src_v.at[pl.ds(c * _B, _B)]],
                                 BD[s], GS[s]),
                pltpu.async_copy(ea_h.at[dst_v.at[pl.ds(c * _B, _B)]],
                                 EA[s], GS[s]),
            ]

        def issue_out(g0, c, s):
            eb = ebase + (g0 + c) * _B
            return [
                pltpu.async_copy(EE[s], r_h.at[pl.ds(eb, _B)], OUTS[s]),
                pltpu.async_copy(BD[s], nd_sh.at[IDST[s]], GS[s], add=True),
            ]

        def compute(s):
            @pl.loop(0, _B, step=4)
            def _(r0):
                for dr in range(4):
                    r = r0 + dr
                    for j in range(4):
                        sl = pl.ds(16 * j, 16)
                        sh = pl.ds(D + 16 * j, 16)
                        bv = BD[s][r, sl]
                        ehat = CE[s][r, sl] + BD[s][r, sh] + EA[s][r, sl]
                        sg = 1.0 / (1.0 + jnp.exp(-ehat))
                        BD[s][r, sl] = sg * bv
                        BD[s][r, sh] = sg
                        EE[s][r, sl] = jnp.maximum(ehat, 0.0)

        @pl.loop(0, _NPH)
        def _(ph):
            g0 = ph * _CPP
            eb0 = ebase + g0 * _B
            d1 = pltpu.async_copy(src_h.at[pl.ds(eb0, _CPP * _B)], src_v,
                                  GS[0])
            d2 = pltpu.async_copy(dst_h.at[pl.ds(eb0, _CPP * _B)], dst_v,
                                  GS[1])
            d1.wait()
            d2.wait()

            @pl.loop(0, _CPP, step=2)
            def _(t):
                din0 = issue_in(g0, t, 0)
                din1 = issue_in(g0, t + 1, 1)
                for d in din0:
                    d.wait()
                compute(0)
                dout0 = issue_out(g0, t, 0)
                for d in din1:
                    d.wait()
                compute(1)
                dout1 = issue_out(g0, t + 1, 1)
                for d in dout0:
                    d.wait()
                for d in dout1:
                    d.wait()

        plsc.subcore_barrier()

        @pl.when(sid == 0)
        def _():
            pltpu.async_copy(nd_sh, nd_h.at[cid], OUTS[0]).wait()

    return k(src, dst, ce, bd, ea)


def _sc_final(src, dst, pe, hahb):
    @functools.partial(
        pl.kernel,
        out_type=jax.ShapeDtypeStruct((E, D), jnp.float32),
        mesh=_sc_mesh(),
        scratch_types=[
            pltpu.VMEM((_CPP * _B,), jnp.int32),
            pltpu.VMEM((_CPP * _B,), jnp.int32),
            pltpu.VMEM((_B, 2 * D), jnp.float32),
            pltpu.VMEM((_B, 2 * D), jnp.float32),
            pltpu.VMEM((_B, 2 * D), jnp.float32),
            pltpu.VMEM((_B, 2 * D), jnp.float32),
            pltpu.VMEM((_B, D), jnp.float32),
            pltpu.VMEM((_B, D), jnp.float32),
            pltpu.SemaphoreType.DMA,
            pltpu.SemaphoreType.DMA,
            pltpu.SemaphoreType.DMA,
            pltpu.SemaphoreType.DMA,
            pltpu.SemaphoreType.DMA,
            pltpu.SemaphoreType.DMA,
        ])
    def k(src_h, dst_h, pe_h, hahb_h, q_h,
          src_v, dst_v, ha0, ha1, hb0, hb1, pe0, pe1,
          ins0, ins1, gs0, gs1, outs0, outs1):
        cid = lax.axis_index("c")
        sid = lax.axis_index("s")
        wid = sid * _NC + cid
        HA = [ha0, ha1]
        HB = [hb0, hb1]
        PE = [pe0, pe1]
        INS = [ins0, ins1]
        GS = [gs0, gs1]
        OUTS = [outs0, outs1]
        ebase = wid * _EPW

        def issue_in(g0, c, s):
            eb = ebase + (g0 + c) * _B
            return [
                pltpu.async_copy(pe_h.at[pl.ds(eb, _B)], PE[s], INS[s]),
                pltpu.async_copy(hahb_h.at[src_v.at[pl.ds(c * _B, _B)]],
                                 HA[s], GS[s]),
                pltpu.async_copy(hahb_h.at[dst_v.at[pl.ds(c * _B, _B)]],
                                 HB[s], GS[s]),
            ]

        def issue_out(g0, c, s):
            eb = ebase + (g0 + c) * _B
            return pltpu.async_copy(PE[s], q_h.at[pl.ds(eb, _B)], OUTS[s])

        def compute(s):
            @pl.loop(0, _B, step=4)
            def _(r0):
                for dr in range(4):
                    r = r0 + dr
                    for j in range(4):
                        sl = pl.ds(16 * j, 16)
                        sh = pl.ds(D + 16 * j, 16)
                        PE[s][r, sl] = jnp.maximum(
                            PE[s][r, sl] + HA[s][r, sl] + HB[s][r, sh], 0.0)

        @pl.loop(0, _NPH)
        def _(ph):
            g0 = ph * _CPP
            eb0 = ebase + g0 * _B
            d1 = pltpu.async_copy(src_h.at[pl.ds(eb0, _CPP * _B)], src_v,
                                  GS[0])
            d2 = pltpu.async_copy(dst_h.at[pl.ds(eb0, _CPP * _B)], dst_v,
                                  GS[1])
            d1.wait()
            d2.wait()

            @pl.loop(0, _CPP, step=2)
            def _(t):
                din0 = issue_in(g0, t, 0)
                din1 = issue_in(g0, t + 1, 1)
                for d in din0:
                    d.wait()
                compute(0)
                dout0 = issue_out(g0, t, 0)
                for d in din1:
                    d.wait()
                compute(1)
                dout1 = issue_out(g0, t + 1, 1)
                dout0.wait()
                dout1.wait()

    return k(src, dst, pe, hahb)


def kernel(edge_index, x, e, params):
    src = edge_index[0]
    dst = edge_index[1]
    p = params
    gnn = p['gnn']

    def wb(layer):
        wcat = jnp.concatenate(
            [layer['B_W'], layer['D_W'], layer['E_W'], layer['A_W']], axis=1)
        bcat = jnp.concatenate(
            [layer['B_b'], layer['D_b'], layer['E_b'], layer['A_b']])
        return wcat, bcat

    wcat0, bcat0 = wb(gnn[0])
    h, bd, ea, ee, ce = _encoders(
        x, p['enc_W1'], p['enc_b1'], p['enc_W2'], p['enc_b2'], wcat0, bcat0,
        e, p['e1_W'], p['e1_b'], p['e2_W'], p['e2_b'],
        gnn[0]['C_W'], gnn[0]['C_b'])

    for li in range(len(gnn) - 1):
        r, ndpart = _sc_layer(src, dst, ce, bd, ea)
        wcat, bcat = wb(gnn[li + 1])
        h, bd, ea, ee, ce = _interlayer(
            h, ea, ndpart, wcat, bcat,
            gnn[li + 1]['C_W'], gnn[li + 1]['C_b'], ee, r)

    r, ndpart = _sc_layer(src, dst, ce, bd, ea)
    w1 = p['sp_W1']
    hahb, pe = _final_prep(
        h, ea, ndpart, jnp.concatenate([w1[:D], w1[D:2 * D]], axis=1),
        w1[2 * D:], p['sp_b1'], ee, r)
    q = _sc_final(src, dst, pe, hahb)
    scores = _matmul_bias(q, p['sp_W2'], p['sp_b2'], 2000)
    return scores


# speedup vs baseline: 248.2906x; 1.0016x over previous
import functools

import jax
import jax.numpy as jnp
from jax import lax
from jax.experimental import pallas as pl
from jax.experimental.pallas import tpu as pltpu
from jax.experimental.pallas import tpu_sc as plsc

N = 10000
E = 320000
D = 64

_INTERPRET = False


def _matmul_bias_body(x_ref, w_ref, b_ref, o_ref):
    o_ref[...] = (
        jnp.dot(x_ref[...], w_ref[...], preferred_element_type=jnp.float32)
        + b_ref[...])


def _matmul_bias(x, w, b, blk):
    rows = x.shape[0]
    f_in, f_out = w.shape
    grid = rows // blk
    return pl.pallas_call(
        _matmul_bias_body,
        grid=(grid,),
        in_specs=[
            pl.BlockSpec((blk, f_in), lambda i: (i, 0)),
            pl.BlockSpec((f_in, f_out), lambda i: (0, 0)),
            pl.BlockSpec((1, f_out), lambda i: (0, 0)),
        ],
        out_specs=pl.BlockSpec((blk, f_out), lambda i: (i, 0)),
        out_shape=jax.ShapeDtypeStruct((rows, f_out), jnp.float32),
        interpret=_INTERPRET,
    )(x, w, b.reshape(1, -1))


_NB = 10
_EBLK = 2000
_NBLK = 1000


def _nodeidx(i):
    return (jnp.minimum(i, _NB - 1), 0)


def _upd(h_ref, eap_ref, nd_ref):
    num = nd_ref[0, :, :D] + nd_ref[1, :, :D]
    den = nd_ref[0, :, D:] + nd_ref[1, :, D:] + 1e-6
    return h_ref[...] + jnp.maximum(eap_ref[:, D:] + num / den, 0.0)


def _interlayer_body(h_ref, eap_ref, nd_ref, wcat_ref, bcat_ref,
                     cw_ref, cb_ref, ee_ref, r_ref,
                     hn_ref, bd_ref, ean_ref, een_ref, ce_ref):
    i = pl.program_id(0)

    @pl.when(i < _NB)
    def _():
        hn = _upd(h_ref, eap_ref, nd_ref)
        hn_ref[...] = hn
        o = (jnp.dot(hn, wcat_ref[...], preferred_element_type=jnp.float32)
             + bcat_ref[...])
        bd_ref[...] = o[:, :2 * D]
        ean_ref[...] = o[:, 2 * D:]

    een = ee_ref[...] + r_ref[...]
    een_ref[...] = een
    ce_ref[...] = (
        jnp.dot(een, cw_ref[...], preferred_element_type=jnp.float32)
        + cb_ref[...])


def _interlayer(h, ea_prev, nd, wcat, bcat, cw, cb, ee, r):
    return pl.pallas_call(
        _interlayer_body,
        grid=(E // _EBLK,),
        in_specs=[
            pl.BlockSpec((_NBLK, D), _nodeidx),
            pl.BlockSpec((_NBLK, 2 * D), _nodeidx),
            pl.BlockSpec((2, _NBLK, 2 * D), lambda i: (0, *_nodeidx(i))),
            pl.BlockSpec((D, 4 * D), lambda i: (0, 0)),
            pl.BlockSpec((1, 4 * D), lambda i: (0, 0)),
            pl.BlockSpec((D, D), lambda i: (0, 0)),
            pl.BlockSpec((1, D), lambda i: (0, 0)),
            pl.BlockSpec((_EBLK, D), lambda i: (i, 0)),
            pl.BlockSpec((_EBLK, D), lambda i: (i, 0)),
        ],
        out_specs=[
            pl.BlockSpec((_NBLK, D), _nodeidx),
            pl.BlockSpec((_NBLK, 2 * D), _nodeidx),
            pl.BlockSpec((_NBLK, 2 * D), _nodeidx),
            pl.BlockSpec((_EBLK, D), lambda i: (i, 0)),
            pl.BlockSpec((_EBLK, D), lambda i: (i, 0)),
        ],
        out_shape=[
            jax.ShapeDtypeStruct((N, D), jnp.float32),
            jax.ShapeDtypeStruct((N, 2 * D), jnp.float32),
            jax.ShapeDtypeStruct((N, 2 * D), jnp.float32),
            jax.ShapeDtypeStruct((E, D), jnp.float32),
            jax.ShapeDtypeStruct((E, D), jnp.float32),
        ],
        interpret=_INTERPRET,
    )(h, ea_prev, nd, wcat, bcat.reshape(1, -1), cw, cb.reshape(1, -1),
      ee, r)


def _final_prep_body(h_ref, eap_ref, nd_ref, w12_ref, w1c_ref, b1_ref,
                     ee_ref, r_ref, hahb_ref, pe_ref):
    i = pl.program_id(0)

    @pl.when(i < _NB)
    def _():
        hn = _upd(h_ref, eap_ref, nd_ref)
        hahb_ref[...] = jnp.dot(hn, w12_ref[...],
                                preferred_element_type=jnp.float32)

    pe_ref[...] = (
        jnp.dot(ee_ref[...] + r_ref[...], w1c_ref[...],
                preferred_element_type=jnp.float32)
        + b1_ref[...])


def _final_prep(h, ea_prev, nd, w12, w1c, b1, ee, r):
    return pl.pallas_call(
        _final_prep_body,
        grid=(E // _EBLK,),
        in_specs=[
            pl.BlockSpec((_NBLK, D), _nodeidx),
            pl.BlockSpec((_NBLK, 2 * D), _nodeidx),
            pl.BlockSpec((2, _NBLK, 2 * D), lambda i: (0, *_nodeidx(i))),
            pl.BlockSpec((D, 2 * D), lambda i: (0, 0)),
            pl.BlockSpec((D, D), lambda i: (0, 0)),
            pl.BlockSpec((1, D), lambda i: (0, 0)),
            pl.BlockSpec((_EBLK, D), lambda i: (i, 0)),
            pl.BlockSpec((_EBLK, D), lambda i: (i, 0)),
        ],
        out_specs=[
            pl.BlockSpec((_NBLK, 2 * D), _nodeidx),
            pl.BlockSpec((_EBLK, D), lambda i: (i, 0)),
        ],
        out_shape=[
            jax.ShapeDtypeStruct((N, 2 * D), jnp.float32),
            jax.ShapeDtypeStruct((E, D), jnp.float32),
        ],
        interpret=_INTERPRET,
    )(h, ea_prev, nd, w12, w1c, b1.reshape(1, -1), ee, r)


def _encoders_body(x_ref, nw1_ref, nb1_ref, nw2_ref, nb2_ref,
                   wcat_ref, bcat_ref,
                   e_ref, ew1_ref, eb1_ref, ew2_ref, eb2_ref,
                   cw_ref, cb_ref,
                   h_ref, bd_ref, ea_ref, ee_ref, ce_ref):
    i = pl.program_id(0)

    @pl.when(i < _NB)
    def _():
        t = jnp.maximum(
            jnp.dot(x_ref[...], nw1_ref[...],
                    preferred_element_type=jnp.float32) + nb1_ref[...], 0.0)
        h = (jnp.dot(t, nw2_ref[...], preferred_element_type=jnp.float32)
             + nb2_ref[...])
        h_ref[...] = h
        o = (jnp.dot(h, wcat_ref[...], preferred_element_type=jnp.float32)
             + bcat_ref[...])
        bd_ref[...] = o[:, :2 * D]
        ea_ref[...] = o[:, 2 * D:]

    t = jnp.maximum(
        jnp.dot(e_ref[...], ew1_ref[...],
                preferred_element_type=jnp.float32) + eb1_ref[...], 0.0)
    ee = (jnp.dot(t, ew2_ref[...], preferred_element_type=jnp.float32)
          + eb2_ref[...])
    ee_ref[...] = ee
    ce_ref[...] = (
        jnp.dot(ee, cw_ref[...], preferred_element_type=jnp.float32)
        + cb_ref[...])


def _encoders(x, nw1, nb1, nw2, nb2, wcat, bcat,
              e, ew1, eb1, ew2, eb2, cw, cb):
    hid = ew1.shape[1]
    return pl.pallas_call(
        _encoders_body,
        grid=(E // _EBLK,),
        in_specs=[
            pl.BlockSpec((_NBLK, x.shape[1]), _nodeidx),
            pl.BlockSpec((x.shape[1], D), lambda i: (0, 0)),
            pl.BlockSpec((1, D), lambda i: (0, 0)),
            pl.BlockSpec((D, D), lambda i: (0, 0)),
            pl.BlockSpec((1, D), lambda i: (0, 0)),
            pl.BlockSpec((D, 4 * D), lambda i: (0, 0)),
            pl.BlockSpec((1, 4 * D), lambda i: (0, 0)),
            pl.BlockSpec((_EBLK, e.shape[1]), lambda i: (i, 0)),
            pl.BlockSpec((e.shape[1], hid), lambda i: (0, 0)),
            pl.BlockSpec((1, hid), lambda i: (0, 0)),
            pl.BlockSpec((hid, D), lambda i: (0, 0)),
            pl.BlockSpec((1, D), lambda i: (0, 0)),
            pl.BlockSpec((D, D), lambda i: (0, 0)),
            pl.BlockSpec((1, D), lambda i: (0, 0)),
        ],
        out_specs=[
            pl.BlockSpec((_NBLK, D), _nodeidx),
            pl.BlockSpec((_NBLK, 2 * D), _nodeidx),
            pl.BlockSpec((_NBLK, 2 * D), _nodeidx),
            pl.BlockSpec((_EBLK, D), lambda i: (i, 0)),
            pl.BlockSpec((_EBLK, D), lambda i: (i, 0)),
        ],
        out_shape=[
            jax.ShapeDtypeStruct((N, D), jnp.float32),
            jax.ShapeDtypeStruct((N, 2 * D), jnp.float32),
            jax.ShapeDtypeStruct((N, 2 * D), jnp.float32),
            jax.ShapeDtypeStruct((E, D), jnp.float32),
            jax.ShapeDtypeStruct((E, D), jnp.float32),
        ],
        interpret=_INTERPRET,
    )(x, nw1, nb1.reshape(1, -1), nw2, nb2.reshape(1, -1),
      wcat, bcat.reshape(1, -1),
      e, ew1, eb1.reshape(1, -1), ew2, eb2.reshape(1, -1),
      cw, cb.reshape(1, -1))


_NC = 2
_NS = 16
_NW = _NC * _NS
_B = 40
_EPW = E // _NW
_TPW = _EPW // _B
_NPH = 5
_CPP = _TPW // _NPH
_NCH = N // _B


def _sc_mesh():
    return plsc.VectorSubcoreMesh(
        core_axis_name="c", subcore_axis_name="s",
        num_cores=_NC, num_subcores=_NS)


def _sc_layer(src, dst, ce, bd, ea):
    @functools.partial(
        pl.kernel,
        out_type=[jax.ShapeDtypeStruct((E, D), jnp.float32),
                  jax.ShapeDtypeStruct((_NC, N, 2 * D), jnp.float32)],
        mesh=_sc_mesh(),
        scratch_types=[
            pltpu.VMEM((_CPP * _B,), jnp.int32),
            pltpu.VMEM((_CPP * _B,), jnp.int32),
            pltpu.VMEM((_B,), jnp.int32),
            pltpu.VMEM((_B,), jnp.int32),
            pltpu.VMEM((_B, 2 * D), jnp.float32),
            pltpu.VMEM((_B, 2 * D), jnp.float32),
            pltpu.VMEM((_B, 2 * D), jnp.float32),
            pltpu.VMEM((_B, 2 * D), jnp.float32),
            pltpu.VMEM((_B, D), jnp.float32),
            pltpu.VMEM((_B, D), jnp.float32),
            pltpu.VMEM((_B, D), jnp.float32),
            pltpu.VMEM((_B, D), jnp.float32),
            pltpu.VMEM_SHARED((N, 2 * D), jnp.float32),
            pltpu.SemaphoreType.DMA,
            pltpu.SemaphoreType.DMA,
            pltpu.SemaphoreType.DMA,
            pltpu.SemaphoreType.DMA,
            pltpu.SemaphoreType.DMA,
            pltpu.SemaphoreType.DMA,
        ])
    def k(src_h, dst_h, ce_h, bd_h, ea_h, r_h, nd_h,
          src_v, dst_v, idst0, idst1, bd0, bd1, ea0, ea1,
          ce0, ce1, ee0, ee1, nd_sh, ins0, ins1, gs0, gs1, outs0, outs1):
        cid = lax.axis_index("c")
        sid = lax.axis_index("s")
        wid = sid * _NC + cid
        IDST = [idst0, idst1]
        BD = [bd0, bd1]
        EA = [ea0, ea1]
        CE = [ce0, ce1]
        EE = [ee0, ee1]
        INS = [ins0, ins1]
        GS = [gs0, gs1]
        OUTS = [outs0, outs1]
        ebase = wid * _EPW

        @pl.loop(0, _B)
        def _(r):
            for j in range(8):
                bd0[r, pl.ds(16 * j, 16)] = jnp.zeros((16,), jnp.float32)

        zd = [pltpu.async_copy(bd0, nd_sh.at[pl.ds((sid + t * _NS) * _B, _B)],
                               INS[0])
              for t in range(_NCH // _NS)]
        for d in zd:
            d.wait()

        @pl.when(sid < _NCH - _NS * (_NCH // _NS))
        def _():
            pltpu.sync_copy(
                bd0, nd_sh.at[pl.ds((sid + (_NCH // _NS) * _NS) * _B, _B)])

        plsc.subcore_barrier()

        def issue_in(g0, c, s):
            eb = ebase + (g0 + c) * _B
            return [
                pltpu.async_copy(ce_h.at[pl.ds(eb, _B)], CE[s], INS[s]),
                pltpu.async_copy(dst_h.at[pl.ds(eb, _B)], IDST[s], INS[s]),
                pltpu.async_copy(bd_h.at[src_v.at[pl.ds(c * _B, _B)]],
                                 BD[s], GS[s]),
                pltpu.async_copy(ea_h.at[dst_v.at[pl.ds(c * _B, _B)]],
                                 EA[s], GS[s]),
            ]

        def issue_out(g0, c, s):
            eb = ebase + (g0 + c) * _B
            return [
                pltpu.async_copy(EE[s], r_h.at[pl.ds(eb, _B)], OUTS[s]),
                pltpu.async_copy(BD[s], nd_sh.at[IDST[s]], GS[s], add=True),
            ]

        def compute(s):
            @pl.loop(0, _B, step=4)
            def _(r0):
                for dr in range(4):
                    r = r0 + dr
                    for j in range(4):
                        sl = pl.ds(16 * j, 16)
                        sh = pl.ds(D + 16 * j, 16)
                        bv = BD[s][r, sl]
                        ehat = CE[s][r, sl] + BD[s][r, sh] + EA[s][r, sl]
                        sg = 1.0 / (1.0 + jnp.exp(-ehat))
                        BD[s][r, sl] = sg * bv
                        BD[s][r, sh] = sg
                        EE[s][r, sl] = jnp.maximum(ehat, 0.0)

        @pl.loop(0, _NPH)
        def _(ph):
            g0 = ph * _CPP
            eb0 = ebase + g0 * _B
            d1 = pltpu.async_copy(src_h.at[pl.ds(eb0, _CPP * _B)], src_v,
                                  GS[0])
            d2 = pltpu.async_copy(dst_h.at[pl.ds(eb0, _CPP * _B)], dst_v,
                                  GS[1])
            d1.wait()
            d2.wait()

            @pl.loop(0, _CPP, step=2)
            def _(t):
                din0 = issue_in(g0, t, 0)
                din1 = issue_in(g0, t + 1, 1)
                for d in din0:
                    d.wait()
                compute(0)
                dout0 = issue_out(g0, t, 0)
                for d in din1:
                    d.wait()
                compute(1)
                dout1 = issue_out(g0, t + 1, 1)
                for d in dout0:
                    d.wait()
                for d in dout1:
                    d.wait()

        plsc.subcore_barrier()

        @pl.when(sid == 0)
        def _():
            pltpu.async_copy(nd_sh, nd_h.at[cid], OUTS[0]).wait()

    return k(src, dst, ce, bd, ea)


def _sc_final(src, dst, pe, hahb):
    @functools.partial(
        pl.kernel,
        out_type=jax.ShapeDtypeStruct((E, D), jnp.float32),
        mesh=_sc_mesh(),
        scratch_types=[
            pltpu.VMEM((_CPP * _B,), jnp.int32),
            pltpu.VMEM((_CPP * _B,), jnp.int32),
            pltpu.VMEM((_B, 2 * D), jnp.float32),
            pltpu.VMEM((_B, 2 * D), jnp.float32),
            pltpu.VMEM((_B, 2 * D), jnp.float32),
            pltpu.VMEM((_B, 2 * D), jnp.float32),
            pltpu.VMEM((_B, D), jnp.float32),
            pltpu.VMEM((_B, D), jnp.float32),
            pltpu.SemaphoreType.DMA,
            pltpu.SemaphoreType.DMA,
            pltpu.SemaphoreType.DMA,
            pltpu.SemaphoreType.DMA,
            pltpu.SemaphoreType.DMA,
            pltpu.SemaphoreType.DMA,
        ])
    def k(src_h, dst_h, pe_h, hahb_h, q_h,
          src_v, dst_v, ha0, ha1, hb0, hb1, pe0, pe1,
          ins0, ins1, gs0, gs1, outs0, outs1):
        cid = lax.axis_index("c")
        sid = lax.axis_index("s")
        wid = sid * _NC + cid
        HA = [ha0, ha1]
        HB = [hb0, hb1]
        PE = [pe0, pe1]
        INS = [ins0, ins1]
        GS = [gs0, gs1]
        OUTS = [outs0, outs1]
        ebase = wid * _EPW

        def issue_in(g0, c, s):
            eb = ebase + (g0 + c) * _B
            return [
                pltpu.async_copy(pe_h.at[pl.ds(eb, _B)], PE[s], INS[s]),
                pltpu.async_copy(hahb_h.at[src_v.at[pl.ds(c * _B, _B)]],
                                 HA[s], GS[s]),
                pltpu.async_copy(hahb_h.at[dst_v.at[pl.ds(c * _B, _B)]],
                                 HB[s], GS[s]),
            ]

        def issue_out(g0, c, s):
            eb = ebase + (g0 + c) * _B
            return pltpu.async_copy(PE[s], q_h.at[pl.ds(eb, _B)], OUTS[s])

        def compute(s):
            @pl.loop(0, _B, step=4)
            def _(r0):
                for dr in range(4):
                    r = r0 + dr
                    for j in range(4):
                        sl = pl.ds(16 * j, 16)
                        sh = pl.ds(D + 16 * j, 16)
                        PE[s][r, sl] = jnp.maximum(
                            PE[s][r, sl] + HA[s][r, sl] + HB[s][r, sh], 0.0)

        @pl.loop(0, _NPH)
        def _(ph):
            g0 = ph * _CPP
            eb0 = ebase + g0 * _B
            d1 = pltpu.async_copy(src_h.at[pl.ds(eb0, _CPP * _B)], src_v,
                                  GS[0])
            d2 = pltpu.async_copy(dst_h.at[pl.ds(eb0, _CPP * _B)], dst_v,
                                  GS[1])
            d1.wait()
            d2.wait()

            @pl.loop(0, _CPP, step=2)
            def _(t):
                din0 = issue_in(g0, t, 0)
                din1 = issue_in(g0, t + 1, 1)
                for d in din0:
                    d.wait()
                compute(0)
                dout0 = issue_out(g0, t, 0)
                for d in din1:
                    d.wait()
                compute(1)
                dout1 = issue_out(g0, t + 1, 1)
                dout0.wait()
                dout1.wait()

    return k(src, dst, pe, hahb)


def kernel(edge_index, x, e, params):
    src = edge_index[0]
    dst = edge_index[1]
    p = params
    gnn = p['gnn']

    def wb(layer):
        wcat = jnp.concatenate(
            [layer['B_W'], layer['D_W'], layer['E_W'], layer['A_W']], axis=1)
        bcat = jnp.concatenate(
            [layer['B_b'], layer['D_b'], layer['E_b'], layer['A_b']])
        return wcat, bcat

    wcat0, bcat0 = wb(gnn[0])
    h, bd, ea, ee, ce = _encoders(
        x, p['enc_W1'], p['enc_b1'], p['enc_W2'], p['enc_b2'], wcat0, bcat0,
        e, p['e1_W'], p['e1_b'], p['e2_W'], p['e2_b'],
        gnn[0]['C_W'], gnn[0]['C_b'])

    for li in range(len(gnn) - 1):
        r, ndpart = _sc_layer(src, dst, ce, bd, ea)
        wcat, bcat = wb(gnn[li + 1])
        h, bd, ea, ee, ce = _interlayer(
            h, ea, ndpart, wcat, bcat,
            gnn[li + 1]['C_W'], gnn[li + 1]['C_b'], ee, r)

    r, ndpart = _sc_layer(src, dst, ce, bd, ea)
    w1 = p['sp_W1']
    hahb, pe = _final_prep(
        h, ea, ndpart, jnp.concatenate([w1[:D], w1[D:2 * D]], axis=1),
        w1[2 * D:], p['sp_b1'], ee, r)
    q = _sc_final(src, dst, pe, hahb)
    scores = _matmul_bias(q, p['sp_W2'], p['sp_b2'], 2000)
    return scores
